# baseline (speedup 1.0000x reference)
import functools

import jax
import jax.numpy as jnp
from jax import lax
from jax.experimental import pallas as pl
from jax.experimental.pallas import tpu as pltpu
from jax.experimental.pallas import tpu_sc as plsc

N = 10000
E = 320000
DN = 128
DE = 16
DG = 128
H = 128

NC = 2
NS = 16
NW = NC * NS
EPW = E // NW
CH = 80
NCH = EPW // CH

NP = 10240
RPT = NP // NS

BIG = 2000
NBIG = E // BIG
BPW = NBIG // NS
NIN = BIG // CH

TN = 1000
TE = 2000


def _pack_rows(p):
    pb = p.astype(jnp.bfloat16).astype(jnp.float32)
    lo = lax.bitcast_convert_type(pb[:, :64], jnp.uint32)
    hi = lax.bitcast_convert_type(pb[:, 64:], jnp.uint32)
    w = (hi & jnp.uint32(0xFFFF0000)) | (lo >> 16)
    return lax.bitcast_convert_type(w, jnp.int32)


def _unpack_rows(wi):
    u = lax.bitcast_convert_type(wi, jnp.uint32)
    hi = lax.bitcast_convert_type(u & jnp.uint32(0xFFFF0000), jnp.float32)
    lo = lax.bitcast_convert_type(u << 16, jnp.float32)
    return lo, hi


def _proj_body(nf, we_s, we_r, g, we_g, be1, wn_g, bn1,
               p_out, q_out, ce_out, cn_out):
    i = pl.program_id(0)
    x = nf[...]
    p_out[...] = _pack_rows(
        jnp.dot(x, we_s[...], preferred_element_type=jnp.float32))
    q_out[...] = _pack_rows(
        jnp.dot(x, we_r[...], preferred_element_type=jnp.float32))

    @pl.when(i == 0)
    def _():
        gv = g[...]
        ce_out[...] = jnp.dot(gv, we_g[...],
                              preferred_element_type=jnp.float32) + be1[...]
        cn_out[...] = jnp.dot(gv, wn_g[...],
                              preferred_element_type=jnp.float32) + bn1[...]


def _projections(nf, we_s, we_r, g, we_g, be1, wn_g, bn1):
    grid = (N // TN,)
    full = lambda shape: pl.BlockSpec(shape, lambda i: (0, 0))
    return pl.pallas_call(
        _proj_body,
        grid=grid,
        in_specs=[
            pl.BlockSpec((TN, DN), lambda i: (i, 0)),
            full((DN, H)), full((DN, H)), full((1, DG)), full((DG, H)),
            full((1, H)), full((DG, H)), full((1, H)),
        ],
        out_specs=[
            pl.BlockSpec((TN, H // 2), lambda i: (i, 0)),
            pl.BlockSpec((TN, H // 2), lambda i: (i, 0)),
            full((1, H)), full((1, H)),
        ],
        out_shape=[
            jax.ShapeDtypeStruct((N, H // 2), jnp.int32),
            jax.ShapeDtypeStruct((N, H // 2), jnp.int32),
            jax.ShapeDtypeStruct((1, H), jnp.float32),
            jax.ShapeDtypeStruct((1, H), jnp.float32),
        ],
    )(nf, we_s, we_r, g, we_g, be1, wn_g, bn1)


def _sc_gather_body(p_hbm, q_hbm, sidx_hbm, ridx_hbm, gp_out, gq_out,
                    sidx_v, ridx_v, rp, rq, sem):
    cid = lax.axis_index("c")
    sid = lax.axis_index("s")
    wid = sid * NC + cid
    pltpu.sync_copy(sidx_hbm.at[wid], sidx_v)
    pltpu.sync_copy(ridx_hbm.at[wid], ridx_v)

    def body(j, carry):
        pltpu.async_copy(p_hbm.at[sidx_v.at[j]], rp, sem).wait()
        pltpu.sync_copy(rp, gp_out.at[wid * NCH + j])
        pltpu.async_copy(q_hbm.at[ridx_v.at[j]], rq, sem).wait()
        pltpu.sync_copy(rq, gq_out.at[wid * NCH + j])
        return carry

    lax.fori_loop(0, NCH, body, 0)


def _sc_gather(p, q, sidx3, ridx3):
    mesh = plsc.VectorSubcoreMesh(core_axis_name="c", subcore_axis_name="s")
    out = pl.kernel(
        _sc_gather_body,
        out_type=[
            jax.ShapeDtypeStruct((E // CH, CH, DN // 2), jnp.int32),
            jax.ShapeDtypeStruct((E // CH, CH, DN // 2), jnp.int32),
        ],
        mesh=mesh,
        scratch_types=[
            pltpu.VMEM((NCH, CH), jnp.int32),
            pltpu.VMEM((NCH, CH), jnp.int32),
            pltpu.VMEM((CH, DN // 2), jnp.int32),
            pltpu.VMEM((CH, DN // 2), jnp.int32),
            pltpu.SemaphoreType.DMA,
        ],
        compiler_params=pltpu.CompilerParams(use_tc_tiling_on_sc=False),
    )(p, q, sidx3, ridx3)
    return out


def _edge_body(ef, gp, gq, we_e, ce, we2, be2, ne_out, e2g_out, acc):
    i = pl.program_id(0)
    p0, p1 = _unpack_rows(gp[...])
    q0, q1 = _unpack_rows(gq[...])
    efv = ef[...]
    we_ev = we_e[...]
    cev = ce[...]
    we2v = we2[...]
    pre0 = (jnp.dot(efv, we_ev[:, :H // 2],
                    preferred_element_type=jnp.float32)
            + p0 + q0 + cev[:, :H // 2])
    pre1 = (jnp.dot(efv, we_ev[:, H // 2:],
                    preferred_element_type=jnp.float32)
            + p1 + q1 + cev[:, H // 2:])
    he0 = jnp.maximum(pre0, 0.0)
    he1 = jnp.maximum(pre1, 0.0)
    ne = (jnp.dot(he0, we2v[:H // 2], preferred_element_type=jnp.float32)
          + jnp.dot(he1, we2v[H // 2:], preferred_element_type=jnp.float32)
          + be2[...])
    ne_out[...] = ne
    part = jnp.sum(ne, axis=0, keepdims=True)

    @pl.when(i == 0)
    def _():
        acc[...] = part

    @pl.when(i > 0)
    def _():
        acc[...] = acc[...] + part

    @pl.when(i == pl.num_programs(0) - 1)
    def _():
        e2g_out[...] = acc[...]


def _edge_mlp(ef, gp, gq, we_e, ce, we2, be2):
    grid = (E // TE,)
    full = lambda shape: pl.BlockSpec(shape, lambda i: (0, 0))
    return pl.pallas_call(
        _edge_body,
        grid=grid,
        in_specs=[
            pl.BlockSpec((TE, DE), lambda i: (i, 0)),
            pl.BlockSpec((TE, DN // 2), lambda i: (i, 0)),
            pl.BlockSpec((TE, DN // 2), lambda i: (i, 0)),
            full((DE, H)), full((1, H)), full((H, DE)), full((1, DE)),
        ],
        out_specs=[
            pl.BlockSpec((TE, DE), lambda i: (i, 0)),
            full((1, DE)),
        ],
        out_shape=[
            jax.ShapeDtypeStruct((E, DE), jnp.float32),
            jax.ShapeDtypeStruct((1, DE), jnp.float32),
        ],
        scratch_shapes=[pltpu.VMEM((1, DE), jnp.float32)],
    )(ef, gp, gq, we_e, ce, we2, be2)


def _sc_scatter_body(ne_hbm, idx_hbm, zer_hbm, agg_out, table, ne_buf, idx_v):
    cid = lax.axis_index("c")
    sid = lax.axis_index("s")
    pltpu.sync_copy(zer_hbm.at[sid], table.at[pl.ds(sid * RPT, RPT)])
    plsc.subcore_barrier()

    def big(t, carry):
        b = sid * BPW + t
        pltpu.sync_copy(ne_hbm.at[b], ne_buf)
        pltpu.sync_copy(idx_hbm.at[cid, b], idx_v)

        def inner(j, c2):
            pltpu.sync_copy(ne_buf.at[pl.ds(j * CH, CH)],
                            table.at[idx_v.at[j]], add=True)
            return c2

        lax.fori_loop(0, NIN, inner, 0)
        return carry

    lax.fori_loop(0, BPW, big, 0)
    plsc.subcore_barrier()
    pltpu.sync_copy(table.at[pl.ds(sid * RPT, RPT)], agg_out.at[cid, sid])


def _sc_scatter(ne3, idx4, zer3):
    mesh = plsc.VectorSubcoreMesh(core_axis_name="c", subcore_axis_name="s")
    return pl.kernel(
        _sc_scatter_body,
        out_type=jax.ShapeDtypeStruct((2, NS, RPT, DE), jnp.float32),
        mesh=mesh,
        scratch_types=[
            pltpu.VMEM_SHARED((NP, DE), jnp.float32),
            pltpu.VMEM((BIG, DE), jnp.float32),
            pltpu.VMEM((NIN, CH), jnp.int32),
        ],
        compiler_params=pltpu.CompilerParams(use_tc_tiling_on_sc=False),
    )(ne3, idx4, zer3)


def _node_body(nf, ags, agr, cn, wn_n, wn_s, wn_r, wn2, bn2,
               g, e2g, wg_g, wg_n, wg_e, bg1, wg2, bg2,
               nn_out, ng_out, nacc):
    i = pl.program_id(0)
    pre = (jnp.dot(nf[...], wn_n[...], preferred_element_type=jnp.float32)
           + jnp.dot(ags[...], wn_s[...], preferred_element_type=jnp.float32)
           + jnp.dot(agr[...], wn_r[...], preferred_element_type=jnp.float32)
           + cn[...])
    hn = jnp.maximum(pre, 0.0)
    nn = jnp.dot(hn, wn2[...], preferred_element_type=jnp.float32) + bn2[...]
    nn_out[...] = nn
    part = jnp.sum(nn, axis=0, keepdims=True)

    @pl.when(i == 0)
    def _():
        nacc[...] = part

    @pl.when(i > 0)
    def _():
        nacc[...] = nacc[...] + part

    @pl.when(i == pl.num_programs(0) - 1)
    def _():
        gpre = (jnp.dot(g[...], wg_g[...], preferred_element_type=jnp.float32)
                + jnp.dot(nacc[...], wg_n[...],
                          preferred_element_type=jnp.float32)
                + jnp.dot(e2g[...], wg_e[...],
                          preferred_element_type=jnp.float32)
                + bg1[...])
        hg = jnp.maximum(gpre, 0.0)
        ng_out[...] = jnp.dot(hg, wg2[...],
                              preferred_element_type=jnp.float32) + bg2[...]


def _node_mlp(nf, ags, agr, cn, wn_n, wn_s, wn_r, wn2, bn2,
              g, e2g, wg_g, wg_n, wg_e, bg1, wg2, bg2):
    grid = (N // TN,)
    full = lambda shape: pl.BlockSpec(shape, lambda i: (0, 0))
    return pl.pallas_call(
        _node_body,
        grid=grid,
        in_specs=[
            pl.BlockSpec((TN, DN), lambda i: (i, 0)),
            pl.BlockSpec((TN, DE), lambda i: (i, 0)),
            pl.BlockSpec((TN, DE), lambda i: (i, 0)),
            full((1, H)), full((DN, H)), full((DE, H)), full((DE, H)),
            full((H, DN)), full((1, DN)),
            full((1, DG)), full((1, DE)),
            full((DG, H)), full((DN, H)), full((DE, H)), full((1, H)),
            full((H, DG)), full((1, DG)),
        ],
        out_specs=[
            pl.BlockSpec((TN, DN), lambda i: (i, 0)),
            full((1, DG)),
        ],
        out_shape=[
            jax.ShapeDtypeStruct((N, DN), jnp.float32),
            jax.ShapeDtypeStruct((1, DG), jnp.float32),
        ],
        scratch_shapes=[pltpu.VMEM((1, DN), jnp.float32)],
    )(nf, ags, agr, cn, wn_n, wn_s, wn_r, wn2, bn2,
      g, e2g, wg_g, wg_n, wg_e, bg1, wg2, bg2)


def kernel(node_features, edge_features, global_features, senders, receivers,
           We1, be1, We2, be2, Wn1, bn1, Wn2, bn2, Wg1, bg1, Wg2, bg2):
    we_e = We1[0:DE]
    we_s = We1[DE:DE + DN]
    we_r = We1[DE + DN:DE + 2 * DN]
    we_g = We1[DE + 2 * DN:]
    wn_n = Wn1[0:DN]
    wn_s = Wn1[DN:DN + DE]
    wn_r = Wn1[DN + DE:DN + 2 * DE]
    wn_g = Wn1[DN + 2 * DE:]
    wg_g = Wg1[0:DG]
    wg_n = Wg1[DG:DG + DN]
    wg_e = Wg1[DG + DN:]

    p, q, ce, cn = _projections(
        node_features, we_s, we_r, global_features, we_g,
        be1.reshape(1, H), wn_g, bn1.reshape(1, H))

    sidx3 = senders.reshape(NW, NCH, CH)
    ridx3 = receivers.reshape(NW, NCH, CH)
    gp3, gq3 = _sc_gather(p, q, sidx3, ridx3)

    new_edges, e2g = _edge_mlp(
        edge_features, gp3.reshape(E, DN // 2), gq3.reshape(E, DN // 2),
        we_e, ce, We2, be2.reshape(1, DE))

    ne3 = new_edges.reshape(NBIG, BIG, DE)
    idx4 = jnp.stack([senders.reshape(NBIG, NIN, CH),
                      receivers.reshape(NBIG, NIN, CH)])
    zer3 = jnp.zeros((NS, RPT, DE), jnp.float32)
    agg4 = _sc_scatter(ne3, idx4, zer3)
    agg = agg4.reshape(2, NP, DE)
    ags = agg[0, :N]
    agr = agg[1, :N]

    new_nodes, new_global = _node_mlp(
        node_features, ags, agr, cn, wn_n, wn_s, wn_r, Wn2,
        bn2.reshape(1, DN), global_features, e2g,
        wg_g, wg_n, wg_e, bg1.reshape(1, H), Wg2, bg2.reshape(1, DG))

    return (new_nodes, new_edges, new_global)


# speedup vs baseline: 1.3131x; 1.3131x over previous
import functools

import jax
import jax.numpy as jnp
from jax import lax
from jax.experimental import pallas as pl
from jax.experimental.pallas import tpu as pltpu
from jax.experimental.pallas import tpu_sc as plsc

N = 10000
E = 320000
DN = 128
DE = 16
DG = 128
H = 128

NC = 2
NS = 16
NW = NC * NS
EPW = E // NW
CH = 80
NCH = EPW // CH

NP = 10240
RPT = NP // NS

BIG = 2000
NBIG = E // BIG
BPW = NBIG // NS
NIN = BIG // CH

TN = 1000
TE = 2560


def _pack_rows(p):
    pb = p.astype(jnp.bfloat16).astype(jnp.float32)
    lo = lax.bitcast_convert_type(pb[:, :64], jnp.uint32)
    hi = lax.bitcast_convert_type(pb[:, 64:], jnp.uint32)
    w = (hi & jnp.uint32(0xFFFF0000)) | (lo >> 16)
    return lax.bitcast_convert_type(w, jnp.int32)


def _unpack_rows(wi):
    u = lax.bitcast_convert_type(wi, jnp.uint32)
    hi = lax.bitcast_convert_type(u & jnp.uint32(0xFFFF0000), jnp.float32)
    lo = lax.bitcast_convert_type(u << 16, jnp.float32)
    return lo, hi


def _proj_body(nf, we_s, we_r, g, we_g, be1, wn_g, bn1,
               p_out, q_out, ce_out, cn_out):
    i = pl.program_id(0)
    x = nf[...]
    p_out[...] = _pack_rows(
        jnp.dot(x, we_s[...], preferred_element_type=jnp.float32))
    q_out[...] = _pack_rows(
        jnp.dot(x, we_r[...], preferred_element_type=jnp.float32))

    @pl.when(i == 0)
    def _():
        gv = g[...]
        ce_out[...] = jnp.dot(gv, we_g[...],
                              preferred_element_type=jnp.float32) + be1[...]
        cn_out[...] = jnp.dot(gv, wn_g[...],
                              preferred_element_type=jnp.float32) + bn1[...]


def _projections(nf, we_s, we_r, g, we_g, be1, wn_g, bn1):
    grid = (N // TN,)
    full = lambda shape: pl.BlockSpec(shape, lambda i: (0, 0))
    return pl.pallas_call(
        _proj_body,
        grid=grid,
        in_specs=[
            pl.BlockSpec((TN, DN), lambda i: (i, 0)),
            full((DN, H)), full((DN, H)), full((1, DG)), full((DG, H)),
            full((1, H)), full((DG, H)), full((1, H)),
        ],
        out_specs=[
            pl.BlockSpec((TN, H // 2), lambda i: (i, 0)),
            pl.BlockSpec((TN, H // 2), lambda i: (i, 0)),
            full((1, H)), full((1, H)),
        ],
        out_shape=[
            jax.ShapeDtypeStruct((N, H // 2), jnp.int32),
            jax.ShapeDtypeStruct((N, H // 2), jnp.int32),
            jax.ShapeDtypeStruct((1, H), jnp.float32),
            jax.ShapeDtypeStruct((1, H), jnp.float32),
        ],
    )(nf, we_s, we_r, g, we_g, be1, wn_g, bn1)


def _sc_gather_body(p_hbm, q_hbm, sidx_hbm, ridx_hbm, gp_out, gq_out,
                    sidx_v, ridx_v, rp, rq, sem):
    cid = lax.axis_index("c")
    sid = lax.axis_index("s")
    wid = sid * NC + cid
    pltpu.sync_copy(sidx_hbm.at[wid], sidx_v)
    pltpu.sync_copy(ridx_hbm.at[wid], ridx_v)

    def body(j, carry):
        pltpu.async_copy(p_hbm.at[sidx_v.at[j]], rp, sem).wait()
        pltpu.sync_copy(rp, gp_out.at[wid * NCH + j])
        pltpu.async_copy(q_hbm.at[ridx_v.at[j]], rq, sem).wait()
        pltpu.sync_copy(rq, gq_out.at[wid * NCH + j])
        return carry

    lax.fori_loop(0, NCH, body, 0)


def _sc_gather(p, q, sidx3, ridx3):
    mesh = plsc.VectorSubcoreMesh(core_axis_name="c", subcore_axis_name="s")
    out = pl.kernel(
        _sc_gather_body,
        out_type=[
            jax.ShapeDtypeStruct((E // CH, CH, DN // 2), jnp.int32),
            jax.ShapeDtypeStruct((E // CH, CH, DN // 2), jnp.int32),
        ],
        mesh=mesh,
        scratch_types=[
            pltpu.VMEM((NCH, CH), jnp.int32),
            pltpu.VMEM((NCH, CH), jnp.int32),
            pltpu.VMEM((CH, DN // 2), jnp.int32),
            pltpu.VMEM((CH, DN // 2), jnp.int32),
            pltpu.SemaphoreType.DMA,
        ],
        compiler_params=pltpu.CompilerParams(use_tc_tiling_on_sc=False),
    )(p, q, sidx3, ridx3)
    return out


PK = 8
TE8 = TE // PK


def _edge_body(ef8, gp8, gq8, wlo, whi, vlo, vhi, clo8, chi8, be28,
               ne_out, e2g_out, acc):
    i = pl.program_id(0)
    plo, phi = _unpack_rows(gp8[...])
    qlo, qhi = _unpack_rows(gq8[...])
    efv = ef8[...]
    prelo = (jnp.dot(efv, wlo[...], preferred_element_type=jnp.float32)
             + plo + qlo + clo8[...])
    prehi = (jnp.dot(efv, whi[...], preferred_element_type=jnp.float32)
             + phi + qhi + chi8[...])
    glo = jnp.maximum(prelo, 0.0)
    ghi = jnp.maximum(prehi, 0.0)
    ne8 = (jnp.dot(glo, vlo[...], preferred_element_type=jnp.float32)
           + jnp.dot(ghi, vhi[...], preferred_element_type=jnp.float32)
           + be28[...])
    ne_out[...] = ne8
    part = jnp.sum(ne8, axis=0, keepdims=True)

    @pl.when(i == 0)
    def _():
        acc[...] = part

    @pl.when(i > 0)
    def _():
        acc[...] = acc[...] + part

    @pl.when(i == pl.num_programs(0) - 1)
    def _():
        a = acc[...]
        s = a[:, 0:DE]
        for k in range(1, PK):
            s = s + a[:, k * DE:(k + 1) * DE]
        e2g_out[...] = s


def _edge_mlp(ef8, gp8, gq8, wlo, whi, vlo, vhi, clo8, chi8, be28):
    grid = (E // TE,)
    full = lambda shape: pl.BlockSpec(shape, lambda i: (0, 0))
    return pl.pallas_call(
        _edge_body,
        grid=grid,
        in_specs=[
            pl.BlockSpec((TE8, PK * DE), lambda i: (i, 0)),
            pl.BlockSpec((TE8, PK * DN // 2), lambda i: (i, 0)),
            pl.BlockSpec((TE8, PK * DN // 2), lambda i: (i, 0)),
            full((PK * DE, PK * H // 2)), full((PK * DE, PK * H // 2)),
            full((PK * H // 2, PK * DE)), full((PK * H // 2, PK * DE)),
            full((1, PK * H // 2)), full((1, PK * H // 2)),
            full((1, PK * DE)),
        ],
        out_specs=[
            pl.BlockSpec((TE8, PK * DE), lambda i: (i, 0)),
            full((1, DE)),
        ],
        out_shape=[
            jax.ShapeDtypeStruct((E // PK, PK * DE), jnp.float32),
            jax.ShapeDtypeStruct((1, DE), jnp.float32),
        ],
        scratch_shapes=[pltpu.VMEM((1, PK * DE), jnp.float32)],
    )(ef8, gp8, gq8, wlo, whi, vlo, vhi, clo8, chi8, be28)


def _sc_scatter_body(ne_hbm, idx_hbm, zer_hbm, agg_out, table, ne_buf, idx_v):
    cid = lax.axis_index("c")
    sid = lax.axis_index("s")
    pltpu.sync_copy(zer_hbm.at[sid], table.at[pl.ds(sid * RPT, RPT)])
    plsc.subcore_barrier()

    def big(t, carry):
        b = sid * BPW + t
        pltpu.sync_copy(ne_hbm.at[b], ne_buf)
        pltpu.sync_copy(idx_hbm.at[cid, b], idx_v)

        def inner(j, c2):
            pltpu.sync_copy(ne_buf.at[pl.ds(j * CH, CH)],
                            table.at[idx_v.at[j]], add=True)
            return c2

        lax.fori_loop(0, NIN, inner, 0)
        return carry

    lax.fori_loop(0, BPW, big, 0)
    plsc.subcore_barrier()
    pltpu.sync_copy(table.at[pl.ds(sid * RPT, RPT)], agg_out.at[cid, sid])


def _sc_scatter(ne3, idx4, zer3):
    mesh = plsc.VectorSubcoreMesh(core_axis_name="c", subcore_axis_name="s")
    return pl.kernel(
        _sc_scatter_body,
        out_type=jax.ShapeDtypeStruct((2, NS, RPT, DE), jnp.float32),
        mesh=mesh,
        scratch_types=[
            pltpu.VMEM_SHARED((NP, DE), jnp.float32),
            pltpu.VMEM((BIG, DE), jnp.float32),
            pltpu.VMEM((NIN, CH), jnp.int32),
        ],
        compiler_params=pltpu.CompilerParams(use_tc_tiling_on_sc=False),
    )(ne3, idx4, zer3)


def _node_body(nf, ags, agr, cn, wn_n, wn_s, wn_r, wn2, bn2,
               g, e2g, wg_g, wg_n, wg_e, bg1, wg2, bg2,
               nn_out, ng_out, nacc):
    i = pl.program_id(0)
    pre = (jnp.dot(nf[...], wn_n[...], preferred_element_type=jnp.float32)
           + jnp.dot(ags[...], wn_s[...], preferred_element_type=jnp.float32)
           + jnp.dot(agr[...], wn_r[...], preferred_element_type=jnp.float32)
           + cn[...])
    hn = jnp.maximum(pre, 0.0)
    nn = jnp.dot(hn, wn2[...], preferred_element_type=jnp.float32) + bn2[...]
    nn_out[...] = nn
    part = jnp.sum(nn, axis=0, keepdims=True)

    @pl.when(i == 0)
    def _():
        nacc[...] = part

    @pl.when(i > 0)
    def _():
        nacc[...] = nacc[...] + part

    @pl.when(i == pl.num_programs(0) - 1)
    def _():
        gpre = (jnp.dot(g[...], wg_g[...], preferred_element_type=jnp.float32)
                + jnp.dot(nacc[...], wg_n[...],
                          preferred_element_type=jnp.float32)
                + jnp.dot(e2g[...], wg_e[...],
                          preferred_element_type=jnp.float32)
                + bg1[...])
        hg = jnp.maximum(gpre, 0.0)
        ng_out[...] = jnp.dot(hg, wg2[...],
                              preferred_element_type=jnp.float32) + bg2[...]


def _node_mlp(nf, ags, agr, cn, wn_n, wn_s, wn_r, wn2, bn2,
              g, e2g, wg_g, wg_n, wg_e, bg1, wg2, bg2):
    grid = (N // TN,)
    full = lambda shape: pl.BlockSpec(shape, lambda i: (0, 0))
    return pl.pallas_call(
        _node_body,
        grid=grid,
        in_specs=[
            pl.BlockSpec((TN, DN), lambda i: (i, 0)),
            pl.BlockSpec((TN, DE), lambda i: (i, 0)),
            pl.BlockSpec((TN, DE), lambda i: (i, 0)),
            full((1, H)), full((DN, H)), full((DE, H)), full((DE, H)),
            full((H, DN)), full((1, DN)),
            full((1, DG)), full((1, DE)),
            full((DG, H)), full((DN, H)), full((DE, H)), full((1, H)),
            full((H, DG)), full((1, DG)),
        ],
        out_specs=[
            pl.BlockSpec((TN, DN), lambda i: (i, 0)),
            full((1, DG)),
        ],
        out_shape=[
            jax.ShapeDtypeStruct((N, DN), jnp.float32),
            jax.ShapeDtypeStruct((1, DG), jnp.float32),
        ],
        scratch_shapes=[pltpu.VMEM((1, DN), jnp.float32)],
    )(nf, ags, agr, cn, wn_n, wn_s, wn_r, wn2, bn2,
      g, e2g, wg_g, wg_n, wg_e, bg1, wg2, bg2)


def kernel(node_features, edge_features, global_features, senders, receivers,
           We1, be1, We2, be2, Wn1, bn1, Wn2, bn2, Wg1, bg1, Wg2, bg2):
    we_e = We1[0:DE]
    we_s = We1[DE:DE + DN]
    we_r = We1[DE + DN:DE + 2 * DN]
    we_g = We1[DE + 2 * DN:]
    wn_n = Wn1[0:DN]
    wn_s = Wn1[DN:DN + DE]
    wn_r = Wn1[DN + DE:DN + 2 * DE]
    wn_g = Wn1[DN + 2 * DE:]
    wg_g = Wg1[0:DG]
    wg_n = Wg1[DG:DG + DN]
    wg_e = Wg1[DG + DN:]

    p, q, ce, cn = _projections(
        node_features, we_s, we_r, global_features, we_g,
        be1.reshape(1, H), wn_g, bn1.reshape(1, H))

    sidx3 = senders.reshape(NW, NCH, CH)
    ridx3 = receivers.reshape(NW, NCH, CH)
    gp3, gq3 = _sc_gather(p, q, sidx3, ridx3)

    eye8 = jnp.eye(PK, dtype=jnp.float32)
    wlo = jnp.kron(eye8, we_e[:, :H // 2])
    whi = jnp.kron(eye8, we_e[:, H // 2:])
    vlo = jnp.kron(eye8, We2[:H // 2])
    vhi = jnp.kron(eye8, We2[H // 2:])
    clo8 = jnp.tile(ce[:, :H // 2], (1, PK))
    chi8 = jnp.tile(ce[:, H // 2:], (1, PK))
    be28 = jnp.tile(be2.reshape(1, DE), (1, PK))

    ne8, e2g = _edge_mlp(
        edge_features.reshape(E // PK, PK * DE),
        gp3.reshape(E // PK, PK * DN // 2),
        gq3.reshape(E // PK, PK * DN // 2),
        wlo, whi, vlo, vhi, clo8, chi8, be28)
    new_edges = ne8.reshape(E, DE)

    ne3 = new_edges.reshape(NBIG, BIG, DE)
    idx4 = jnp.stack([senders.reshape(NBIG, NIN, CH),
                      receivers.reshape(NBIG, NIN, CH)])
    zer3 = jnp.zeros((NS, RPT, DE), jnp.float32)
    agg4 = _sc_scatter(ne3, idx4, zer3)
    agg = agg4.reshape(2, NP, DE)
    ags = agg[0, :N]
    agr = agg[1, :N]

    new_nodes, new_global = _node_mlp(
        node_features, ags, agr, cn, wn_n, wn_s, wn_r, Wn2,
        bn2.reshape(1, DN), global_features, e2g,
        wg_g, wg_n, wg_e, bg1.reshape(1, H), Wg2, bg2.reshape(1, DG))

    return (new_nodes, new_edges, new_global)


# speedup vs baseline: 1.5366x; 1.1702x over previous
import functools

import jax
import jax.numpy as jnp
from jax import lax
from jax.experimental import pallas as pl
from jax.experimental.pallas import tpu as pltpu
from jax.experimental.pallas import tpu_sc as plsc

N = 10000
E = 320000
DN = 128
DE = 16
DG = 128
H = 128

NC = 2
NS = 16
NW = NC * NS
EPW = E // NW
CH = 80
NCH = EPW // CH

GCH = 40
GNCH = EPW // GCH
GB = 5
GIT = GNCH // (2 * GB)

NP = 10240
RPT = NP // NS

BIG = 2000
NBIG = E // BIG
BPW = NBIG // NS
NIN = BIG // CH

TN = 1000
TE = 2560


def _pack_rows(p):
    pb = p.astype(jnp.bfloat16).astype(jnp.float32)
    lo = lax.bitcast_convert_type(pb[:, :64], jnp.uint32)
    hi = lax.bitcast_convert_type(pb[:, 64:], jnp.uint32)
    w = (hi & jnp.uint32(0xFFFF0000)) | (lo >> 16)
    return lax.bitcast_convert_type(w, jnp.int32)


def _unpack_rows(wi):
    u = lax.bitcast_convert_type(wi, jnp.uint32)
    hi = lax.bitcast_convert_type(u & jnp.uint32(0xFFFF0000), jnp.float32)
    lo = lax.bitcast_convert_type(u << 16, jnp.float32)
    return lo, hi


def _proj_body(nf, we_s, we_r, g, we_g, be1, wn_g, bn1,
               p_out, q_out, ce_out, cn_out):
    i = pl.program_id(0)
    x = nf[...]
    p_out[...] = _pack_rows(
        jnp.dot(x, we_s[...], preferred_element_type=jnp.float32))
    q_out[...] = _pack_rows(
        jnp.dot(x, we_r[...], preferred_element_type=jnp.float32))

    @pl.when(i == 0)
    def _():
        gv = g[...]
        ce_out[...] = jnp.dot(gv, we_g[...],
                              preferred_element_type=jnp.float32) + be1[...]
        cn_out[...] = jnp.dot(gv, wn_g[...],
                              preferred_element_type=jnp.float32) + bn1[...]


def _projections(nf, we_s, we_r, g, we_g, be1, wn_g, bn1):
    grid = (N // TN,)
    full = lambda shape: pl.BlockSpec(shape, lambda i: (0, 0))
    return pl.pallas_call(
        _proj_body,
        grid=grid,
        in_specs=[
            pl.BlockSpec((TN, DN), lambda i: (i, 0)),
            full((DN, H)), full((DN, H)), full((1, DG)), full((DG, H)),
            full((1, H)), full((DG, H)), full((1, H)),
        ],
        out_specs=[
            pl.BlockSpec((TN, H // 2), lambda i: (i, 0)),
            pl.BlockSpec((TN, H // 2), lambda i: (i, 0)),
            full((1, H)), full((1, H)),
        ],
        out_shape=[
            jax.ShapeDtypeStruct((N, H // 2), jnp.int32),
            jax.ShapeDtypeStruct((N, H // 2), jnp.int32),
            jax.ShapeDtypeStruct((1, H), jnp.float32),
            jax.ShapeDtypeStruct((1, H), jnp.float32),
        ],
    )(nf, we_s, we_r, g, we_g, be1, wn_g, bn1)


def _sc_gather_body(p_hbm, q_hbm, sidx_hbm, ridx_hbm, gp_out, gq_out,
                    sidx_v, ridx_v, bufp, bufq,
                    gspa, gsqa, gspb, gsqb, wspa, wsqa, wspb, wsqb):
    cid = lax.axis_index("c")
    sid = lax.axis_index("s")
    wid = sid * NC + cid
    base_row = wid * GNCH
    pltpu.sync_copy(sidx_hbm.at[wid], sidx_v)
    pltpu.sync_copy(ridx_hbm.at[wid], ridx_v)

    def fire_gathers(c0, s0, semp, semq):
        for b in range(GB):
            pltpu.async_copy(p_hbm.at[sidx_v.at[c0 + b]], bufp.at[s0 + b],
                             semp)
            pltpu.async_copy(q_hbm.at[ridx_v.at[c0 + b]], bufq.at[s0 + b],
                             semq)

    def drain_gathers(c0, s0, semp, semq):
        for b in range(GB):
            pltpu.make_async_copy(p_hbm.at[sidx_v.at[c0 + b]],
                                  bufp.at[s0 + b], semp).wait()
            pltpu.make_async_copy(q_hbm.at[ridx_v.at[c0 + b]],
                                  bufq.at[s0 + b], semq).wait()

    def fire_wbs(c0, s0, semp, semq):
        for b in range(GB):
            pltpu.async_copy(bufp.at[s0 + b], gp_out.at[base_row + c0 + b],
                             semp)
            pltpu.async_copy(bufq.at[s0 + b], gq_out.at[base_row + c0 + b],
                             semq)

    def drain_wbs(c0, s0, semp, semq):
        for b in range(GB):
            pltpu.make_async_copy(bufp.at[s0 + b],
                                  gp_out.at[base_row + c0 + b], semp).wait()
            pltpu.make_async_copy(bufq.at[s0 + b],
                                  gq_out.at[base_row + c0 + b], semq).wait()

    fire_gathers(0, 0, gspa, gsqa)

    def body(it, carry):
        base = it * 2 * GB

        @pl.when(it > 0)
        def _():
            drain_wbs(base - GB, GB, wspb, wsqb)

        fire_gathers(base + GB, GB, gspb, gsqb)
        drain_gathers(base, 0, gspa, gsqa)
        fire_wbs(base, 0, wspa, wsqa)

        @pl.when(it < GIT - 1)
        def _():
            drain_wbs(base, 0, wspa, wsqa)
            fire_gathers(base + 2 * GB, 0, gspa, gsqa)

        drain_gathers(base + GB, GB, gspb, gsqb)
        fire_wbs(base + GB, GB, wspb, wsqb)
        return carry

    lax.fori_loop(0, GIT, body, 0)
    last = (GIT - 1) * 2 * GB
    drain_wbs(last, 0, wspa, wsqa)
    drain_wbs(last + GB, GB, wspb, wsqb)


def _sc_gather(p, q, sidx3, ridx3):
    mesh = plsc.VectorSubcoreMesh(core_axis_name="c", subcore_axis_name="s")
    out = pl.kernel(
        _sc_gather_body,
        out_type=[
            jax.ShapeDtypeStruct((E // GCH, GCH, DN // 2), jnp.int32),
            jax.ShapeDtypeStruct((E // GCH, GCH, DN // 2), jnp.int32),
        ],
        mesh=mesh,
        scratch_types=[
            pltpu.VMEM((GNCH, GCH), jnp.int32),
            pltpu.VMEM((GNCH, GCH), jnp.int32),
            pltpu.VMEM((2 * GB, GCH, DN // 2), jnp.int32),
            pltpu.VMEM((2 * GB, GCH, DN // 2), jnp.int32),
        ] + [pltpu.SemaphoreType.DMA] * 8,
        compiler_params=pltpu.CompilerParams(use_tc_tiling_on_sc=False),
    )(p, q, sidx3, ridx3)
    return out


PK = 8
TE8 = TE // PK


def _edge_body(ef8, gp8, gq8, wlo, whi, vlo, vhi, clo8, chi8, be28,
               ne_out, e2g_out, acc):
    i = pl.program_id(0)
    plo, phi = _unpack_rows(gp8[...])
    qlo, qhi = _unpack_rows(gq8[...])
    efv = ef8[...]
    prelo = (jnp.dot(efv, wlo[...], preferred_element_type=jnp.float32)
             + plo + qlo + clo8[...])
    prehi = (jnp.dot(efv, whi[...], preferred_element_type=jnp.float32)
             + phi + qhi + chi8[...])
    glo = jnp.maximum(prelo, 0.0)
    ghi = jnp.maximum(prehi, 0.0)
    ne8 = (jnp.dot(glo, vlo[...], preferred_element_type=jnp.float32)
           + jnp.dot(ghi, vhi[...], preferred_element_type=jnp.float32)
           + be28[...])
    ne_out[...] = ne8
    part = jnp.sum(ne8, axis=0, keepdims=True)

    @pl.when(i == 0)
    def _():
        acc[...] = part

    @pl.when(i > 0)
    def _():
        acc[...] = acc[...] + part

    @pl.when(i == pl.num_programs(0) - 1)
    def _():
        a = acc[...]
        s = a[:, 0:DE]
        for k in range(1, PK):
            s = s + a[:, k * DE:(k + 1) * DE]
        e2g_out[...] = s


def _edge_mlp(ef8, gp8, gq8, wlo, whi, vlo, vhi, clo8, chi8, be28):
    grid = (E // TE,)
    full = lambda shape: pl.BlockSpec(shape, lambda i: (0, 0))
    return pl.pallas_call(
        _edge_body,
        grid=grid,
        in_specs=[
            pl.BlockSpec((TE8, PK * DE), lambda i: (i, 0)),
            pl.BlockSpec((TE8, PK * DN // 2), lambda i: (i, 0)),
            pl.BlockSpec((TE8, PK * DN // 2), lambda i: (i, 0)),
            full((PK * DE, PK * H // 2)), full((PK * DE, PK * H // 2)),
            full((PK * H // 2, PK * DE)), full((PK * H // 2, PK * DE)),
            full((1, PK * H // 2)), full((1, PK * H // 2)),
            full((1, PK * DE)),
        ],
        out_specs=[
            pl.BlockSpec((TE8, PK * DE), lambda i: (i, 0)),
            full((1, DE)),
        ],
        out_shape=[
            jax.ShapeDtypeStruct((E // PK, PK * DE), jnp.float32),
            jax.ShapeDtypeStruct((1, DE), jnp.float32),
        ],
        scratch_shapes=[pltpu.VMEM((1, PK * DE), jnp.float32)],
    )(ef8, gp8, gq8, wlo, whi, vlo, vhi, clo8, chi8, be28)


def _sc_scatter_body(ne_hbm, idx_hbm, zer_hbm, agg_out, table, ne_buf, idx_v):
    cid = lax.axis_index("c")
    sid = lax.axis_index("s")
    pltpu.sync_copy(zer_hbm.at[sid], table.at[pl.ds(sid * RPT, RPT)])
    plsc.subcore_barrier()

    def big(t, carry):
        b = sid * BPW + t
        pltpu.sync_copy(ne_hbm.at[b], ne_buf)
        pltpu.sync_copy(idx_hbm.at[cid, b], idx_v)

        def inner(j, c2):
            pltpu.sync_copy(ne_buf.at[pl.ds(j * CH, CH)],
                            table.at[idx_v.at[j]], add=True)
            return c2

        lax.fori_loop(0, NIN, inner, 0)
        return carry

    lax.fori_loop(0, BPW, big, 0)
    plsc.subcore_barrier()
    pltpu.sync_copy(table.at[pl.ds(sid * RPT, RPT)], agg_out.at[cid, sid])


def _sc_scatter(ne3, idx4, zer3):
    mesh = plsc.VectorSubcoreMesh(core_axis_name="c", subcore_axis_name="s")
    return pl.kernel(
        _sc_scatter_body,
        out_type=jax.ShapeDtypeStruct((2, NS, RPT, DE), jnp.float32),
        mesh=mesh,
        scratch_types=[
            pltpu.VMEM_SHARED((NP, DE), jnp.float32),
            pltpu.VMEM((BIG, DE), jnp.float32),
            pltpu.VMEM((NIN, CH), jnp.int32),
        ],
        compiler_params=pltpu.CompilerParams(use_tc_tiling_on_sc=False),
    )(ne3, idx4, zer3)


def _node_body(nf, ags, agr, cn, wn_n, wn_s, wn_r, wn2, bn2,
               g, e2g, wg_g, wg_n, wg_e, bg1, wg2, bg2,
               nn_out, ng_out, nacc):
    i = pl.program_id(0)
    pre = (jnp.dot(nf[...], wn_n[...], preferred_element_type=jnp.float32)
           + jnp.dot(ags[...], wn_s[...], preferred_element_type=jnp.float32)
           + jnp.dot(agr[...], wn_r[...], preferred_element_type=jnp.float32)
           + cn[...])
    hn = jnp.maximum(pre, 0.0)
    nn = jnp.dot(hn, wn2[...], preferred_element_type=jnp.float32) + bn2[...]
    nn_out[...] = nn
    part = jnp.sum(nn, axis=0, keepdims=True)

    @pl.when(i == 0)
    def _():
        nacc[...] = part

    @pl.when(i > 0)
    def _():
        nacc[...] = nacc[...] + part

    @pl.when(i == pl.num_programs(0) - 1)
    def _():
        gpre = (jnp.dot(g[...], wg_g[...], preferred_element_type=jnp.float32)
                + jnp.dot(nacc[...], wg_n[...],
                          preferred_element_type=jnp.float32)
                + jnp.dot(e2g[...], wg_e[...],
                          preferred_element_type=jnp.float32)
                + bg1[...])
        hg = jnp.maximum(gpre, 0.0)
        ng_out[...] = jnp.dot(hg, wg2[...],
                              preferred_element_type=jnp.float32) + bg2[...]


def _node_mlp(nf, ags, agr, cn, wn_n, wn_s, wn_r, wn2, bn2,
              g, e2g, wg_g, wg_n, wg_e, bg1, wg2, bg2):
    grid = (N // TN,)
    full = lambda shape: pl.BlockSpec(shape, lambda i: (0, 0))
    return pl.pallas_call(
        _node_body,
        grid=grid,
        in_specs=[
            pl.BlockSpec((TN, DN), lambda i: (i, 0)),
            pl.BlockSpec((TN, DE), lambda i: (i, 0)),
            pl.BlockSpec((TN, DE), lambda i: (i, 0)),
            full((1, H)), full((DN, H)), full((DE, H)), full((DE, H)),
            full((H, DN)), full((1, DN)),
            full((1, DG)), full((1, DE)),
            full((DG, H)), full((DN, H)), full((DE, H)), full((1, H)),
            full((H, DG)), full((1, DG)),
        ],
        out_specs=[
            pl.BlockSpec((TN, DN), lambda i: (i, 0)),
            full((1, DG)),
        ],
        out_shape=[
            jax.ShapeDtypeStruct((N, DN), jnp.float32),
            jax.ShapeDtypeStruct((1, DG), jnp.float32),
        ],
        scratch_shapes=[pltpu.VMEM((1, DN), jnp.float32)],
    )(nf, ags, agr, cn, wn_n, wn_s, wn_r, wn2, bn2,
      g, e2g, wg_g, wg_n, wg_e, bg1, wg2, bg2)


def kernel(node_features, edge_features, global_features, senders, receivers,
           We1, be1, We2, be2, Wn1, bn1, Wn2, bn2, Wg1, bg1, Wg2, bg2):
    we_e = We1[0:DE]
    we_s = We1[DE:DE + DN]
    we_r = We1[DE + DN:DE + 2 * DN]
    we_g = We1[DE + 2 * DN:]
    wn_n = Wn1[0:DN]
    wn_s = Wn1[DN:DN + DE]
    wn_r = Wn1[DN + DE:DN + 2 * DE]
    wn_g = Wn1[DN + 2 * DE:]
    wg_g = Wg1[0:DG]
    wg_n = Wg1[DG:DG + DN]
    wg_e = Wg1[DG + DN:]

    p, q, ce, cn = _projections(
        node_features, we_s, we_r, global_features, we_g,
        be1.reshape(1, H), wn_g, bn1.reshape(1, H))

    sidx3 = senders.reshape(NW, GNCH, GCH)
    ridx3 = receivers.reshape(NW, GNCH, GCH)
    gp3, gq3 = _sc_gather(p, q, sidx3, ridx3)

    eye8 = jnp.eye(PK, dtype=jnp.float32)
    wlo = jnp.kron(eye8, we_e[:, :H // 2])
    whi = jnp.kron(eye8, we_e[:, H // 2:])
    vlo = jnp.kron(eye8, We2[:H // 2])
    vhi = jnp.kron(eye8, We2[H // 2:])
    clo8 = jnp.tile(ce[:, :H // 2], (1, PK))
    chi8 = jnp.tile(ce[:, H // 2:], (1, PK))
    be28 = jnp.tile(be2.reshape(1, DE), (1, PK))

    ne8, e2g = _edge_mlp(
        edge_features.reshape(E // PK, PK * DE),
        gp3.reshape(E // PK, PK * DN // 2),
        gq3.reshape(E // PK, PK * DN // 2),
        wlo, whi, vlo, vhi, clo8, chi8, be28)
    new_edges = ne8.reshape(E, DE)

    ne3 = new_edges.reshape(NBIG, BIG, DE)
    idx4 = jnp.stack([senders.reshape(NBIG, NIN, CH),
                      receivers.reshape(NBIG, NIN, CH)])
    zer3 = jnp.zeros((NS, RPT, DE), jnp.float32)
    agg4 = _sc_scatter(ne3, idx4, zer3)
    agg = agg4.reshape(2, NP, DE)
    ags = agg[0, :N]
    agr = agg[1, :N]

    new_nodes, new_global = _node_mlp(
        node_features, ags, agr, cn, wn_n, wn_s, wn_r, Wn2,
        bn2.reshape(1, DN), global_features, e2g,
        wg_g, wg_n, wg_e, bg1.reshape(1, H), Wg2, bg2.reshape(1, DG))

    return (new_nodes, new_edges, new_global)


# speedup vs baseline: 1.5653x; 1.0187x over previous
import functools

import jax
import jax.numpy as jnp
from jax import lax
from jax.experimental import pallas as pl
from jax.experimental.pallas import tpu as pltpu
from jax.experimental.pallas import tpu_sc as plsc

N = 10000
E = 320000
DN = 128
DE = 16
DG = 128
H = 128

NC = 2
NS = 16
NW = NC * NS
EPW = E // NW
CH = 80
NCH = EPW // CH

GCH = 40
GNCH = EPW // GCH
GB = 5
GIT = GNCH // (2 * GB)

NP = 10240
RPT = NP // NS

BIG = 2000
NBIG = E // BIG
BPW = NBIG // NS
NIN = BIG // CH

TN = 1000
TE = 2560


def _pack_rows(p):
    pb = p.astype(jnp.bfloat16).astype(jnp.float32)
    lo = lax.bitcast_convert_type(pb[:, :64], jnp.uint32)
    hi = lax.bitcast_convert_type(pb[:, 64:], jnp.uint32)
    w = (hi & jnp.uint32(0xFFFF0000)) | (lo >> 16)
    return lax.bitcast_convert_type(w, jnp.int32)


def _unpack_rows(wi):
    u = lax.bitcast_convert_type(wi, jnp.uint32)
    hi = lax.bitcast_convert_type(u & jnp.uint32(0xFFFF0000), jnp.float32)
    lo = lax.bitcast_convert_type(u << 16, jnp.float32)
    return lo, hi


def _proj_body(nf, we_s, we_r, g, we_g, be1, wn_g, bn1,
               p_out, q_out, ce_out, cn_out):
    i = pl.program_id(0)
    x = nf[...]
    p_out[...] = _pack_rows(
        jnp.dot(x, we_s[...], preferred_element_type=jnp.float32))
    q_out[...] = _pack_rows(
        jnp.dot(x, we_r[...], preferred_element_type=jnp.float32))

    @pl.when(i == 0)
    def _():
        gv = g[...]
        ce_out[...] = jnp.dot(gv, we_g[...],
                              preferred_element_type=jnp.float32) + be1[...]
        cn_out[...] = jnp.dot(gv, wn_g[...],
                              preferred_element_type=jnp.float32) + bn1[...]


def _projections(nf, we_s, we_r, g, we_g, be1, wn_g, bn1):
    grid = (N // TN,)
    full = lambda shape: pl.BlockSpec(shape, lambda i: (0, 0))
    return pl.pallas_call(
        _proj_body,
        grid=grid,
        in_specs=[
            pl.BlockSpec((TN, DN), lambda i: (i, 0)),
            full((DN, H)), full((DN, H)), full((1, DG)), full((DG, H)),
            full((1, H)), full((DG, H)), full((1, H)),
        ],
        out_specs=[
            pl.BlockSpec((TN, H // 2), lambda i: (i, 0)),
            pl.BlockSpec((TN, H // 2), lambda i: (i, 0)),
            full((1, H)), full((1, H)),
        ],
        out_shape=[
            jax.ShapeDtypeStruct((N, H // 2), jnp.int32),
            jax.ShapeDtypeStruct((N, H // 2), jnp.int32),
            jax.ShapeDtypeStruct((1, H), jnp.float32),
            jax.ShapeDtypeStruct((1, H), jnp.float32),
        ],
    )(nf, we_s, we_r, g, we_g, be1, wn_g, bn1)


def _sc_gather_body(p_hbm, q_hbm, sidx_hbm, ridx_hbm, gp_out, gq_out,
                    sidx_v, ridx_v, bufp, bufq,
                    gspa, gsqa, gspb, gsqb, wspa, wsqa, wspb, wsqb):
    cid = lax.axis_index("c")
    sid = lax.axis_index("s")
    wid = sid * NC + cid
    base_row = wid * GNCH
    pltpu.sync_copy(sidx_hbm.at[wid], sidx_v)
    pltpu.sync_copy(ridx_hbm.at[wid], ridx_v)

    def fire_gathers(c0, s0, semp, semq):
        for b in range(GB):
            pltpu.async_copy(p_hbm.at[sidx_v.at[c0 + b]], bufp.at[s0 + b],
                             semp)
            pltpu.async_copy(q_hbm.at[ridx_v.at[c0 + b]], bufq.at[s0 + b],
                             semq)

    def drain_gathers(c0, s0, semp, semq):
        for b in range(GB):
            pltpu.make_async_copy(p_hbm.at[sidx_v.at[c0 + b]],
                                  bufp.at[s0 + b], semp).wait()
            pltpu.make_async_copy(q_hbm.at[ridx_v.at[c0 + b]],
                                  bufq.at[s0 + b], semq).wait()

    def fire_wbs(c0, s0, semp, semq):
        for b in range(GB):
            pltpu.async_copy(bufp.at[s0 + b], gp_out.at[base_row + c0 + b],
                             semp)
            pltpu.async_copy(bufq.at[s0 + b], gq_out.at[base_row + c0 + b],
                             semq)

    def drain_wbs(c0, s0, semp, semq):
        for b in range(GB):
            pltpu.make_async_copy(bufp.at[s0 + b],
                                  gp_out.at[base_row + c0 + b], semp).wait()
            pltpu.make_async_copy(bufq.at[s0 + b],
                                  gq_out.at[base_row + c0 + b], semq).wait()

    fire_gathers(0, 0, gspa, gsqa)

    def body(it, carry):
        base = it * 2 * GB

        @pl.when(it > 0)
        def _():
            drain_wbs(base - GB, GB, wspb, wsqb)

        fire_gathers(base + GB, GB, gspb, gsqb)
        drain_gathers(base, 0, gspa, gsqa)
        fire_wbs(base, 0, wspa, wsqa)

        @pl.when(it < GIT - 1)
        def _():
            drain_wbs(base, 0, wspa, wsqa)
            fire_gathers(base + 2 * GB, 0, gspa, gsqa)

        drain_gathers(base + GB, GB, gspb, gsqb)
        fire_wbs(base + GB, GB, wspb, wsqb)
        return carry

    lax.fori_loop(0, GIT, body, 0)
    last = (GIT - 1) * 2 * GB
    drain_wbs(last, 0, wspa, wsqa)
    drain_wbs(last + GB, GB, wspb, wsqb)


def _sc_gather(p, q, sidx3, ridx3):
    mesh = plsc.VectorSubcoreMesh(core_axis_name="c", subcore_axis_name="s")
    out = pl.kernel(
        _sc_gather_body,
        out_type=[
            jax.ShapeDtypeStruct((E // GCH, GCH, DN // 2), jnp.int32),
            jax.ShapeDtypeStruct((E // GCH, GCH, DN // 2), jnp.int32),
        ],
        mesh=mesh,
        scratch_types=[
            pltpu.VMEM((GNCH, GCH), jnp.int32),
            pltpu.VMEM((GNCH, GCH), jnp.int32),
            pltpu.VMEM((2 * GB, GCH, DN // 2), jnp.int32),
            pltpu.VMEM((2 * GB, GCH, DN // 2), jnp.int32),
        ] + [pltpu.SemaphoreType.DMA] * 8,
        compiler_params=pltpu.CompilerParams(use_tc_tiling_on_sc=False),
    )(p, q, sidx3, ridx3)
    return out


PK = 8
TE8 = TE // PK


def _edge_body(ef8, gp8, gq8, wlo, whi, vlo, vhi, clo8, chi8, be28,
               ne_out, e2g_out, acc):
    i = pl.program_id(0)
    plo, phi = _unpack_rows(gp8[...])
    qlo, qhi = _unpack_rows(gq8[...])
    efv = ef8[...]
    prelo = (jnp.dot(efv, wlo[...], preferred_element_type=jnp.float32)
             + plo + qlo + clo8[...])
    prehi = (jnp.dot(efv, whi[...], preferred_element_type=jnp.float32)
             + phi + qhi + chi8[...])
    glo = jnp.maximum(prelo, 0.0)
    ghi = jnp.maximum(prehi, 0.0)
    ne8 = (jnp.dot(glo, vlo[...], preferred_element_type=jnp.float32)
           + jnp.dot(ghi, vhi[...], preferred_element_type=jnp.float32)
           + be28[...])
    ne_out[...] = ne8
    part = jnp.sum(ne8, axis=0, keepdims=True)

    @pl.when(i == 0)
    def _():
        acc[...] = part

    @pl.when(i > 0)
    def _():
        acc[...] = acc[...] + part

    @pl.when(i == pl.num_programs(0) - 1)
    def _():
        a = acc[...]
        s = a[:, 0:DE]
        for k in range(1, PK):
            s = s + a[:, k * DE:(k + 1) * DE]
        e2g_out[...] = s


def _edge_mlp(ef8, gp8, gq8, wlo, whi, vlo, vhi, clo8, chi8, be28):
    grid = (E // TE,)
    full = lambda shape: pl.BlockSpec(shape, lambda i: (0, 0))
    return pl.pallas_call(
        _edge_body,
        grid=grid,
        in_specs=[
            pl.BlockSpec((TE8, PK * DE), lambda i: (i, 0)),
            pl.BlockSpec((TE8, PK * DN // 2), lambda i: (i, 0)),
            pl.BlockSpec((TE8, PK * DN // 2), lambda i: (i, 0)),
            full((PK * DE, PK * H // 2)), full((PK * DE, PK * H // 2)),
            full((PK * H // 2, PK * DE)), full((PK * H // 2, PK * DE)),
            full((1, PK * H // 2)), full((1, PK * H // 2)),
            full((1, PK * DE)),
        ],
        out_specs=[
            pl.BlockSpec((TE8, PK * DE), lambda i: (i, 0)),
            full((1, DE)),
        ],
        out_shape=[
            jax.ShapeDtypeStruct((E // PK, PK * DE), jnp.float32),
            jax.ShapeDtypeStruct((1, DE), jnp.float32),
        ],
        scratch_shapes=[pltpu.VMEM((1, PK * DE), jnp.float32)],
    )(ef8, gp8, gq8, wlo, whi, vlo, vhi, clo8, chi8, be28)


SBW = NBIG // 2 // NS


def _sc_scatter_body(ne_hbm, sidx_hbm, ridx_hbm, agg_out,
                     tabs, tabr, ne_buf, sidx_v, ridx_v,
                     sem_ne, sem_si, sem_ri, sem_add):
    cid = lax.axis_index("c")
    sid = lax.axis_index("s")

    zrow = jnp.zeros((DE,), jnp.float32)

    def zloop(i, c):
        ne_buf[0, i, :] = zrow
        return c

    lax.fori_loop(0, RPT, zloop, 0)
    pltpu.sync_copy(ne_buf.at[0, pl.ds(0, RPT)], tabs.at[pl.ds(sid * RPT, RPT)])
    pltpu.sync_copy(ne_buf.at[0, pl.ds(0, RPT)], tabr.at[pl.ds(sid * RPT, RPT)])
    plsc.subcore_barrier()

    base_big = cid * (NBIG // 2) + sid * SBW

    def fire_loads(t, p):
        pltpu.async_copy(ne_hbm.at[base_big + t], ne_buf.at[p], sem_ne)
        pltpu.async_copy(sidx_hbm.at[base_big + t], sidx_v.at[p], sem_si)
        pltpu.async_copy(ridx_hbm.at[base_big + t], ridx_v.at[p], sem_ri)

    def drain_loads(t, p):
        pltpu.make_async_copy(ne_hbm.at[base_big + t], ne_buf.at[p],
                              sem_ne).wait()
        pltpu.make_async_copy(sidx_hbm.at[base_big + t], sidx_v.at[p],
                              sem_si).wait()
        pltpu.make_async_copy(ridx_hbm.at[base_big + t], ridx_v.at[p],
                              sem_ri).wait()

    fire_loads(0, 0)

    def big(t, carry):
        p = lax.rem(t, 2)
        drain_loads(t, p)
        for j in range(NIN):
            pltpu.async_copy(ne_buf.at[p, pl.ds(j * CH, CH)],
                             tabs.at[sidx_v.at[p, j]], sem_add, add=True)
            pltpu.async_copy(ne_buf.at[p, pl.ds(j * CH, CH)],
                             tabr.at[ridx_v.at[p, j]], sem_add, add=True)

        @pl.when(t < SBW - 1)
        def _():
            fire_loads(t + 1, 1 - p)

        for j in range(NIN):
            pltpu.make_async_copy(ne_buf.at[p, pl.ds(j * CH, CH)],
                                  tabs.at[sidx_v.at[p, j]], sem_add).wait()
            pltpu.make_async_copy(ne_buf.at[p, pl.ds(j * CH, CH)],
                                  tabr.at[ridx_v.at[p, j]], sem_add).wait()
        return carry

    lax.fori_loop(0, SBW, big, 0)
    plsc.subcore_barrier()
    pltpu.sync_copy(tabs.at[pl.ds(sid * RPT, RPT)], agg_out.at[cid, 0, sid])
    pltpu.sync_copy(tabr.at[pl.ds(sid * RPT, RPT)], agg_out.at[cid, 1, sid])


def _sc_scatter(ne3, sidxb, ridxb):
    mesh = plsc.VectorSubcoreMesh(core_axis_name="c", subcore_axis_name="s")
    return pl.kernel(
        _sc_scatter_body,
        out_type=jax.ShapeDtypeStruct((2, 2, NS, RPT, DE), jnp.float32),
        mesh=mesh,
        scratch_types=[
            pltpu.VMEM_SHARED((NP, DE), jnp.float32),
            pltpu.VMEM_SHARED((NP, DE), jnp.float32),
            pltpu.VMEM((2, BIG, DE), jnp.float32),
            pltpu.VMEM((2, NIN, CH), jnp.int32),
            pltpu.VMEM((2, NIN, CH), jnp.int32),
        ] + [pltpu.SemaphoreType.DMA] * 4,
        compiler_params=pltpu.CompilerParams(use_tc_tiling_on_sc=False),
    )(ne3, sidxb, ridxb)


def _node_body(nf, ags, agr, cn, wn_n, wn_s, wn_r, wn2, bn2,
               g, e2g, wg_g, wg_n, wg_e, bg1, wg2, bg2,
               nn_out, ng_out, nacc):
    i = pl.program_id(0)
    pre = (jnp.dot(nf[...], wn_n[...], preferred_element_type=jnp.float32)
           + jnp.dot(ags[...], wn_s[...], preferred_element_type=jnp.float32)
           + jnp.dot(agr[...], wn_r[...], preferred_element_type=jnp.float32)
           + cn[...])
    hn = jnp.maximum(pre, 0.0)
    nn = jnp.dot(hn, wn2[...], preferred_element_type=jnp.float32) + bn2[...]
    nn_out[...] = nn
    part = jnp.sum(nn, axis=0, keepdims=True)

    @pl.when(i == 0)
    def _():
        nacc[...] = part

    @pl.when(i > 0)
    def _():
        nacc[...] = nacc[...] + part

    @pl.when(i == pl.num_programs(0) - 1)
    def _():
        gpre = (jnp.dot(g[...], wg_g[...], preferred_element_type=jnp.float32)
                + jnp.dot(nacc[...], wg_n[...],
                          preferred_element_type=jnp.float32)
                + jnp.dot(e2g[...], wg_e[...],
                          preferred_element_type=jnp.float32)
                + bg1[...])
        hg = jnp.maximum(gpre, 0.0)
        ng_out[...] = jnp.dot(hg, wg2[...],
                              preferred_element_type=jnp.float32) + bg2[...]


def _node_mlp(nf, ags, agr, cn, wn_n, wn_s, wn_r, wn2, bn2,
              g, e2g, wg_g, wg_n, wg_e, bg1, wg2, bg2):
    grid = (N // TN,)
    full = lambda shape: pl.BlockSpec(shape, lambda i: (0, 0))
    return pl.pallas_call(
        _node_body,
        grid=grid,
        in_specs=[
            pl.BlockSpec((TN, DN), lambda i: (i, 0)),
            pl.BlockSpec((TN, DE), lambda i: (i, 0)),
            pl.BlockSpec((TN, DE), lambda i: (i, 0)),
            full((1, H)), full((DN, H)), full((DE, H)), full((DE, H)),
            full((H, DN)), full((1, DN)),
            full((1, DG)), full((1, DE)),
            full((DG, H)), full((DN, H)), full((DE, H)), full((1, H)),
            full((H, DG)), full((1, DG)),
        ],
        out_specs=[
            pl.BlockSpec((TN, DN), lambda i: (i, 0)),
            full((1, DG)),
        ],
        out_shape=[
            jax.ShapeDtypeStruct((N, DN), jnp.float32),
            jax.ShapeDtypeStruct((1, DG), jnp.float32),
        ],
        scratch_shapes=[pltpu.VMEM((1, DN), jnp.float32)],
    )(nf, ags, agr, cn, wn_n, wn_s, wn_r, wn2, bn2,
      g, e2g, wg_g, wg_n, wg_e, bg1, wg2, bg2)


def kernel(node_features, edge_features, global_features, senders, receivers,
           We1, be1, We2, be2, Wn1, bn1, Wn2, bn2, Wg1, bg1, Wg2, bg2):
    we_e = We1[0:DE]
    we_s = We1[DE:DE + DN]
    we_r = We1[DE + DN:DE + 2 * DN]
    we_g = We1[DE + 2 * DN:]
    wn_n = Wn1[0:DN]
    wn_s = Wn1[DN:DN + DE]
    wn_r = Wn1[DN + DE:DN + 2 * DE]
    wn_g = Wn1[DN + 2 * DE:]
    wg_g = Wg1[0:DG]
    wg_n = Wg1[DG:DG + DN]
    wg_e = Wg1[DG + DN:]

    p, q, ce, cn = _projections(
        node_features, we_s, we_r, global_features, we_g,
        be1.reshape(1, H), wn_g, bn1.reshape(1, H))

    sidx3 = senders.reshape(NW, GNCH, GCH)
    ridx3 = receivers.reshape(NW, GNCH, GCH)
    gp3, gq3 = _sc_gather(p, q, sidx3, ridx3)

    eye8 = jnp.eye(PK, dtype=jnp.float32)
    wlo = jnp.kron(eye8, we_e[:, :H // 2])
    whi = jnp.kron(eye8, we_e[:, H // 2:])
    vlo = jnp.kron(eye8, We2[:H // 2])
    vhi = jnp.kron(eye8, We2[H // 2:])
    clo8 = jnp.tile(ce[:, :H // 2], (1, PK))
    chi8 = jnp.tile(ce[:, H // 2:], (1, PK))
    be28 = jnp.tile(be2.reshape(1, DE), (1, PK))

    ne8, e2g = _edge_mlp(
        edge_features.reshape(E // PK, PK * DE),
        gp3.reshape(E // PK, PK * DN // 2),
        gq3.reshape(E // PK, PK * DN // 2),
        wlo, whi, vlo, vhi, clo8, chi8, be28)
    new_edges = ne8.reshape(E, DE)

    ne3 = new_edges.reshape(NBIG, BIG, DE)
    agg5 = _sc_scatter(ne3, senders.reshape(NBIG, NIN, CH),
                       receivers.reshape(NBIG, NIN, CH))
    agg = agg5.reshape(2, 2, NP, DE)
    ags = agg[0, 0, :N] + agg[1, 0, :N]
    agr = agg[0, 1, :N] + agg[1, 1, :N]

    new_nodes, new_global = _node_mlp(
        node_features, ags, agr, cn, wn_n, wn_s, wn_r, Wn2,
        bn2.reshape(1, DN), global_features, e2g,
        wg_g, wg_n, wg_e, bg1.reshape(1, H), Wg2, bg2.reshape(1, DG))

    return (new_nodes, new_edges, new_global)


# speedup vs baseline: 1.6904x; 1.0799x over previous
import functools

import jax
import jax.numpy as jnp
from jax import lax
from jax.experimental import pallas as pl
from jax.experimental.pallas import tpu as pltpu
from jax.experimental.pallas import tpu_sc as plsc

N = 10000
E = 320000
DN = 128
DE = 16
DG = 128
H = 128

NC = 2
NS = 16
NW = NC * NS
EPW = E // NW
CH = 80
NCH = EPW // CH

GCH = 40
GNCH = EPW // GCH
GB = 5
GIT = GNCH // (2 * GB)

NP = 10240
RPT = NP // NS

BIG = 2000
NBIG = E // BIG
BPW = NBIG // NS
NIN = BIG // CH

TN = 1000
TE = 6400


def _pack_rows(p):
    pb = p.astype(jnp.bfloat16).astype(jnp.float32)
    lo = lax.bitcast_convert_type(pb[:, :64], jnp.uint32)
    hi = lax.bitcast_convert_type(pb[:, 64:], jnp.uint32)
    w = (hi & jnp.uint32(0xFFFF0000)) | (lo >> 16)
    return lax.bitcast_convert_type(w, jnp.int32)


def _unpack_rows(wi):
    u = lax.bitcast_convert_type(wi, jnp.uint32)
    hi = lax.bitcast_convert_type(u & jnp.uint32(0xFFFF0000), jnp.float32)
    lo = lax.bitcast_convert_type(u << 16, jnp.float32)
    return lo, hi


def _proj_body(nf, we_s, we_r, g, we_g, be1, wn_g, bn1,
               p_out, q_out, ce_out, cn_out):
    i = pl.program_id(0)
    x = nf[...]
    p_out[...] = _pack_rows(
        jnp.dot(x, we_s[...], preferred_element_type=jnp.float32))
    q_out[...] = _pack_rows(
        jnp.dot(x, we_r[...], preferred_element_type=jnp.float32))

    @pl.when(i == 0)
    def _():
        gv = g[...]
        ce_out[...] = jnp.dot(gv, we_g[...],
                              preferred_element_type=jnp.float32) + be1[...]
        cn_out[...] = jnp.dot(gv, wn_g[...],
                              preferred_element_type=jnp.float32) + bn1[...]


def _projections(nf, we_s, we_r, g, we_g, be1, wn_g, bn1):
    grid = (N // TN,)
    full = lambda shape: pl.BlockSpec(shape, lambda i: (0, 0))
    return pl.pallas_call(
        _proj_body,
        grid=grid,
        in_specs=[
            pl.BlockSpec((TN, DN), lambda i: (i, 0)),
            full((DN, H)), full((DN, H)), full((1, DG)), full((DG, H)),
            full((1, H)), full((DG, H)), full((1, H)),
        ],
        out_specs=[
            pl.BlockSpec((TN, H // 2), lambda i: (i, 0)),
            pl.BlockSpec((TN, H // 2), lambda i: (i, 0)),
            full((1, H)), full((1, H)),
        ],
        out_shape=[
            jax.ShapeDtypeStruct((N, H // 2), jnp.int32),
            jax.ShapeDtypeStruct((N, H // 2), jnp.int32),
            jax.ShapeDtypeStruct((1, H), jnp.float32),
            jax.ShapeDtypeStruct((1, H), jnp.float32),
        ],
    )(nf, we_s, we_r, g, we_g, be1, wn_g, bn1)


def _sc_gather_body(p_hbm, q_hbm, sidx_hbm, ridx_hbm, gp_out, gq_out,
                    sidx_v, ridx_v, bufp, bufq,
                    gspa, gsqa, gspb, gsqb, wspa, wsqa, wspb, wsqb):
    cid = lax.axis_index("c")
    sid = lax.axis_index("s")
    wid = sid * NC + cid
    base_row = wid * GNCH
    pltpu.sync_copy(sidx_hbm.at[wid], sidx_v)
    pltpu.sync_copy(ridx_hbm.at[wid], ridx_v)

    def fire_gathers(c0, s0, semp, semq):
        for b in range(GB):
            pltpu.async_copy(p_hbm.at[sidx_v.at[c0 + b]], bufp.at[s0 + b],
                             semp)
            pltpu.async_copy(q_hbm.at[ridx_v.at[c0 + b]], bufq.at[s0 + b],
                             semq)

    def drain_gathers(c0, s0, semp, semq):
        for b in range(GB):
            pltpu.make_async_copy(p_hbm.at[sidx_v.at[c0 + b]],
                                  bufp.at[s0 + b], semp).wait()
            pltpu.make_async_copy(q_hbm.at[ridx_v.at[c0 + b]],
                                  bufq.at[s0 + b], semq).wait()

    def fire_wbs(c0, s0, semp, semq):
        for b in range(GB):
            pltpu.async_copy(bufp.at[s0 + b], gp_out.at[base_row + c0 + b],
                             semp)
            pltpu.async_copy(bufq.at[s0 + b], gq_out.at[base_row + c0 + b],
                             semq)

    def drain_wbs(c0, s0, semp, semq):
        for b in range(GB):
            pltpu.make_async_copy(bufp.at[s0 + b],
                                  gp_out.at[base_row + c0 + b], semp).wait()
            pltpu.make_async_copy(bufq.at[s0 + b],
                                  gq_out.at[base_row + c0 + b], semq).wait()

    fire_gathers(0, 0, gspa, gsqa)

    def body(it, carry):
        base = it * 2 * GB

        @pl.when(it > 0)
        def _():
            drain_wbs(base - GB, GB, wspb, wsqb)

        fire_gathers(base + GB, GB, gspb, gsqb)
        drain_gathers(base, 0, gspa, gsqa)
        fire_wbs(base, 0, wspa, wsqa)

        @pl.when(it < GIT - 1)
        def _():
            drain_wbs(base, 0, wspa, wsqa)
            fire_gathers(base + 2 * GB, 0, gspa, gsqa)

        drain_gathers(base + GB, GB, gspb, gsqb)
        fire_wbs(base + GB, GB, wspb, wsqb)
        return carry

    lax.fori_loop(0, GIT, body, 0)
    last = (GIT - 1) * 2 * GB
    drain_wbs(last, 0, wspa, wsqa)
    drain_wbs(last + GB, GB, wspb, wsqb)


def _sc_gather(p, q, sidx3, ridx3):
    mesh = plsc.VectorSubcoreMesh(core_axis_name="c", subcore_axis_name="s")
    out = pl.kernel(
        _sc_gather_body,
        out_type=[
            jax.ShapeDtypeStruct((E // GCH, GCH, DN // 2), jnp.int32),
            jax.ShapeDtypeStruct((E // GCH, GCH, DN // 2), jnp.int32),
        ],
        mesh=mesh,
        scratch_types=[
            pltpu.VMEM((GNCH, GCH), jnp.int32),
            pltpu.VMEM((GNCH, GCH), jnp.int32),
            pltpu.VMEM((2 * GB, GCH, DN // 2), jnp.int32),
            pltpu.VMEM((2 * GB, GCH, DN // 2), jnp.int32),
        ] + [pltpu.SemaphoreType.DMA] * 8,
        compiler_params=pltpu.CompilerParams(use_tc_tiling_on_sc=False),
    )(p, q, sidx3, ridx3)
    return out


PK = 8
TE8 = TE // PK


def _edge_body(ef8, gp8, gq8, wlo, whi, vlo, vhi, clo8, chi8, be28,
               ne_out, e2g_out, acc):
    i = pl.program_id(0)
    plo, phi = _unpack_rows(gp8[...])
    qlo, qhi = _unpack_rows(gq8[...])
    efv = ef8[...]
    prelo = (jnp.dot(efv, wlo[...], preferred_element_type=jnp.float32)
             + plo + qlo + clo8[...])
    prehi = (jnp.dot(efv, whi[...], preferred_element_type=jnp.float32)
             + phi + qhi + chi8[...])
    glo = jnp.maximum(prelo, 0.0)
    ghi = jnp.maximum(prehi, 0.0)
    ne8 = (jnp.dot(glo, vlo[...], preferred_element_type=jnp.float32)
           + jnp.dot(ghi, vhi[...], preferred_element_type=jnp.float32)
           + be28[...])
    ne_out[...] = ne8
    part = jnp.sum(ne8, axis=0, keepdims=True)

    @pl.when(i == 0)
    def _():
        acc[...] = part

    @pl.when(i > 0)
    def _():
        acc[...] = acc[...] + part

    @pl.when(i == pl.num_programs(0) - 1)
    def _():
        a = acc[...]
        s = a[:, 0:DE]
        for k in range(1, PK):
            s = s + a[:, k * DE:(k + 1) * DE]
        e2g_out[...] = s


def _edge_mlp(ef8, gp8, gq8, wlo, whi, vlo, vhi, clo8, chi8, be28):
    grid = (E // TE,)
    full = lambda shape: pl.BlockSpec(shape, lambda i: (0, 0))
    return pl.pallas_call(
        _edge_body,
        grid=grid,
        in_specs=[
            pl.BlockSpec((TE8, PK * DE), lambda i: (i, 0)),
            pl.BlockSpec((TE8, PK * DN // 2), lambda i: (i, 0)),
            pl.BlockSpec((TE8, PK * DN // 2), lambda i: (i, 0)),
            full((PK * DE, PK * H // 2)), full((PK * DE, PK * H // 2)),
            full((PK * H // 2, PK * DE)), full((PK * H // 2, PK * DE)),
            full((1, PK * H // 2)), full((1, PK * H // 2)),
            full((1, PK * DE)),
        ],
        out_specs=[
            pl.BlockSpec((TE8, PK * DE), lambda i: (i, 0)),
            full((1, DE)),
        ],
        out_shape=[
            jax.ShapeDtypeStruct((E // PK, PK * DE), jnp.float32),
            jax.ShapeDtypeStruct((1, DE), jnp.float32),
        ],
        scratch_shapes=[pltpu.VMEM((1, PK * DE), jnp.float32)],
    )(ef8, gp8, gq8, wlo, whi, vlo, vhi, clo8, chi8, be28)


SBW = NBIG // 2 // NS


def _sc_scatter_body(ne_hbm, sidx_hbm, ridx_hbm, agg_out,
                     tabs, tabr, ne_buf, sidx_v, ridx_v,
                     sem_ne, sem_si, sem_ri, sem_add):
    cid = lax.axis_index("c")
    sid = lax.axis_index("s")

    zrow = jnp.zeros((DE,), jnp.float32)

    def zloop(i, c):
        ne_buf[0, i, :] = zrow
        return c

    lax.fori_loop(0, RPT, zloop, 0)
    pltpu.sync_copy(ne_buf.at[0, pl.ds(0, RPT)], tabs.at[pl.ds(sid * RPT, RPT)])
    pltpu.sync_copy(ne_buf.at[0, pl.ds(0, RPT)], tabr.at[pl.ds(sid * RPT, RPT)])
    plsc.subcore_barrier()

    base_big = cid * (NBIG // 2) + sid * SBW

    def fire_loads(t, p):
        pltpu.async_copy(ne_hbm.at[base_big + t], ne_buf.at[p], sem_ne)
        pltpu.async_copy(sidx_hbm.at[base_big + t], sidx_v.at[p], sem_si)
        pltpu.async_copy(ridx_hbm.at[base_big + t], ridx_v.at[p], sem_ri)

    def drain_loads(t, p):
        pltpu.make_async_copy(ne_hbm.at[base_big + t], ne_buf.at[p],
                              sem_ne).wait()
        pltpu.make_async_copy(sidx_hbm.at[base_big + t], sidx_v.at[p],
                              sem_si).wait()
        pltpu.make_async_copy(ridx_hbm.at[base_big + t], ridx_v.at[p],
                              sem_ri).wait()

    fire_loads(0, 0)

    def big(t, carry):
        p = lax.rem(t, 2)
        drain_loads(t, p)
        for j in range(NIN):
            pltpu.async_copy(ne_buf.at[p, pl.ds(j * CH, CH)],
                             tabs.at[sidx_v.at[p, j]], sem_add, add=True)
            pltpu.async_copy(ne_buf.at[p, pl.ds(j * CH, CH)],
                             tabr.at[ridx_v.at[p, j]], sem_add, add=True)

        @pl.when(t < SBW - 1)
        def _():
            fire_loads(t + 1, 1 - p)

        for j in range(NIN):
            pltpu.make_async_copy(ne_buf.at[p, pl.ds(j * CH, CH)],
                                  tabs.at[sidx_v.at[p, j]], sem_add).wait()
            pltpu.make_async_copy(ne_buf.at[p, pl.ds(j * CH, CH)],
                                  tabr.at[ridx_v.at[p, j]], sem_add).wait()
        return carry

    lax.fori_loop(0, SBW, big, 0)
    plsc.subcore_barrier()
    pltpu.sync_copy(tabs.at[pl.ds(sid * RPT, RPT)], agg_out.at[cid, 0, sid])
    pltpu.sync_copy(tabr.at[pl.ds(sid * RPT, RPT)], agg_out.at[cid, 1, sid])


def _sc_scatter(ne3, sidxb, ridxb):
    mesh = plsc.VectorSubcoreMesh(core_axis_name="c", subcore_axis_name="s")
    return pl.kernel(
        _sc_scatter_body,
        out_type=jax.ShapeDtypeStruct((2, 2, NS, RPT, DE), jnp.float32),
        mesh=mesh,
        scratch_types=[
            pltpu.VMEM_SHARED((NP, DE), jnp.float32),
            pltpu.VMEM_SHARED((NP, DE), jnp.float32),
            pltpu.VMEM((2, BIG, DE), jnp.float32),
            pltpu.VMEM((2, NIN, CH), jnp.int32),
            pltpu.VMEM((2, NIN, CH), jnp.int32),
        ] + [pltpu.SemaphoreType.DMA] * 4,
        compiler_params=pltpu.CompilerParams(use_tc_tiling_on_sc=False),
    )(ne3, sidxb, ridxb)


def _node_body(nf, ag0, ag1, ag2, ag3, cn, wn_n, wn_s, wn_r, wn2, bn2,
               g, e2g, wg_g, wg_n, wg_e, bg1, wg2, bg2,
               nn_out, ng_out, nacc):
    i = pl.program_id(0)
    ags = ag0[0] + ag1[0]
    agr = ag2[0] + ag3[0]
    pre = (jnp.dot(nf[...], wn_n[...], preferred_element_type=jnp.float32)
           + jnp.dot(ags, wn_s[...], preferred_element_type=jnp.float32)
           + jnp.dot(agr, wn_r[...], preferred_element_type=jnp.float32)
           + cn[...])
    hn = jnp.maximum(pre, 0.0)
    nn = jnp.dot(hn, wn2[...], preferred_element_type=jnp.float32) + bn2[...]
    nn_out[...] = nn
    part = jnp.sum(nn, axis=0, keepdims=True)

    @pl.when(i == 0)
    def _():
        nacc[...] = part

    @pl.when(i > 0)
    def _():
        nacc[...] = nacc[...] + part

    @pl.when(i == pl.num_programs(0) - 1)
    def _():
        gpre = (jnp.dot(g[...], wg_g[...], preferred_element_type=jnp.float32)
                + jnp.dot(nacc[...], wg_n[...],
                          preferred_element_type=jnp.float32)
                + jnp.dot(e2g[...], wg_e[...],
                          preferred_element_type=jnp.float32)
                + bg1[...])
        hg = jnp.maximum(gpre, 0.0)
        ng_out[...] = jnp.dot(hg, wg2[...],
                              preferred_element_type=jnp.float32) + bg2[...]


def _node_mlp(nf, agg2, cn, wn_n, wn_s, wn_r, wn2, bn2,
              g, e2g, wg_g, wg_n, wg_e, bg1, wg2, bg2):
    grid = (N // TN,)
    full = lambda shape: pl.BlockSpec(shape, lambda i: (0, 0))
    aspec = [pl.BlockSpec((1, TN, DE), lambda i, k=k: (k, i, 0))
             for k in (0, 2, 1, 3)]
    return pl.pallas_call(
        _node_body,
        grid=grid,
        in_specs=[
            pl.BlockSpec((TN, DN), lambda i: (i, 0)),
            *aspec,
            full((1, H)), full((DN, H)), full((DE, H)), full((DE, H)),
            full((H, DN)), full((1, DN)),
            full((1, DG)), full((1, DE)),
            full((DG, H)), full((DN, H)), full((DE, H)), full((1, H)),
            full((H, DG)), full((1, DG)),
        ],
        out_specs=[
            pl.BlockSpec((TN, DN), lambda i: (i, 0)),
            full((1, DG)),
        ],
        out_shape=[
            jax.ShapeDtypeStruct((N, DN), jnp.float32),
            jax.ShapeDtypeStruct((1, DG), jnp.float32),
        ],
        scratch_shapes=[pltpu.VMEM((1, DN), jnp.float32)],
    )(nf, agg2, agg2, agg2, agg2, cn, wn_n, wn_s, wn_r, wn2, bn2,
      g, e2g, wg_g, wg_n, wg_e, bg1, wg2, bg2)


def kernel(node_features, edge_features, global_features, senders, receivers,
           We1, be1, We2, be2, Wn1, bn1, Wn2, bn2, Wg1, bg1, Wg2, bg2):
    we_e = We1[0:DE]
    we_s = We1[DE:DE + DN]
    we_r = We1[DE + DN:DE + 2 * DN]
    we_g = We1[DE + 2 * DN:]
    wn_n = Wn1[0:DN]
    wn_s = Wn1[DN:DN + DE]
    wn_r = Wn1[DN + DE:DN + 2 * DE]
    wn_g = Wn1[DN + 2 * DE:]
    wg_g = Wg1[0:DG]
    wg_n = Wg1[DG:DG + DN]
    wg_e = Wg1[DG + DN:]

    p, q, ce, cn = _projections(
        node_features, we_s, we_r, global_features, we_g,
        be1.reshape(1, H), wn_g, bn1.reshape(1, H))

    sidx3 = senders.reshape(NW, GNCH, GCH)
    ridx3 = receivers.reshape(NW, GNCH, GCH)
    gp3, gq3 = _sc_gather(p, q, sidx3, ridx3)

    eye8 = jnp.eye(PK, dtype=jnp.float32)
    wlo = jnp.kron(eye8, we_e[:, :H // 2])
    whi = jnp.kron(eye8, we_e[:, H // 2:])
    vlo = jnp.kron(eye8, We2[:H // 2])
    vhi = jnp.kron(eye8, We2[H // 2:])
    clo8 = jnp.tile(ce[:, :H // 2], (1, PK))
    chi8 = jnp.tile(ce[:, H // 2:], (1, PK))
    be28 = jnp.tile(be2.reshape(1, DE), (1, PK))

    ne8, e2g = _edge_mlp(
        edge_features.reshape(E // PK, PK * DE),
        gp3.reshape(E // PK, PK * DN // 2),
        gq3.reshape(E // PK, PK * DN // 2),
        wlo, whi, vlo, vhi, clo8, chi8, be28)
    new_edges = ne8.reshape(E, DE)

    ne3 = new_edges.reshape(NBIG, BIG, DE)
    agg5 = _sc_scatter(ne3, senders.reshape(NBIG, NIN, CH),
                       receivers.reshape(NBIG, NIN, CH))
    agg2 = agg5.reshape(4, NP, DE)

    new_nodes, new_global = _node_mlp(
        node_features, agg2, cn, wn_n, wn_s, wn_r, Wn2,
        bn2.reshape(1, DN), global_features, e2g,
        wg_g, wg_n, wg_e, bg1.reshape(1, H), Wg2, bg2.reshape(1, DG))

    return (new_nodes, new_edges, new_global)


# speedup vs baseline: 1.7373x; 1.0278x over previous
import functools

import jax
import jax.numpy as jnp
from jax import lax
from jax.experimental import pallas as pl
from jax.experimental.pallas import tpu as pltpu
from jax.experimental.pallas import tpu_sc as plsc

N = 10000
E = 320000
DN = 128
DE = 16
DG = 128
H = 128

NC = 2
NS = 16
NW = NC * NS
EPW = E // NW
CH = 80
NCH = EPW // CH

GCH = 40
GNCH = EPW // GCH
GB = 5
GIT = GNCH // (2 * GB)

NP = 10240
RPT = NP // NS

BIG = 2000
NBIG = E // BIG
BPW = NBIG // NS
NIN = BIG // CH

TN = 2000
TE = 12800


def _pack_rows(p):
    pb = p.astype(jnp.bfloat16).astype(jnp.float32)
    lo = lax.bitcast_convert_type(pb[:, :64], jnp.uint32)
    hi = lax.bitcast_convert_type(pb[:, 64:], jnp.uint32)
    w = (hi & jnp.uint32(0xFFFF0000)) | (lo >> 16)
    return lax.bitcast_convert_type(w, jnp.int32)


def _unpack_rows(wi):
    u = lax.bitcast_convert_type(wi, jnp.uint32)
    hi = lax.bitcast_convert_type(u & jnp.uint32(0xFFFF0000), jnp.float32)
    lo = lax.bitcast_convert_type(u << 16, jnp.float32)
    return lo, hi


def _proj_body(nf, we_s, we_r, g, we_g, be1, wn_g, bn1,
               p_out, q_out, ce_out, cn_out):
    i = pl.program_id(0)
    x = nf[...]
    p_out[...] = _pack_rows(
        jnp.dot(x, we_s[...], preferred_element_type=jnp.float32))
    q_out[...] = _pack_rows(
        jnp.dot(x, we_r[...], preferred_element_type=jnp.float32))

    @pl.when(i == 0)
    def _():
        gv = g[...]
        ce_out[...] = jnp.dot(gv, we_g[...],
                              preferred_element_type=jnp.float32) + be1[...]
        cn_out[...] = jnp.dot(gv, wn_g[...],
                              preferred_element_type=jnp.float32) + bn1[...]


def _projections(nf, we_s, we_r, g, we_g, be1, wn_g, bn1):
    grid = (N // TN,)
    full = lambda shape: pl.BlockSpec(shape, lambda i: (0, 0))
    return pl.pallas_call(
        _proj_body,
        grid=grid,
        in_specs=[
            pl.BlockSpec((TN, DN), lambda i: (i, 0)),
            full((DN, H)), full((DN, H)), full((1, DG)), full((DG, H)),
            full((1, H)), full((DG, H)), full((1, H)),
        ],
        out_specs=[
            pl.BlockSpec((TN, H // 2), lambda i: (i, 0)),
            pl.BlockSpec((TN, H // 2), lambda i: (i, 0)),
            full((1, H)), full((1, H)),
        ],
        out_shape=[
            jax.ShapeDtypeStruct((N, H // 2), jnp.int32),
            jax.ShapeDtypeStruct((N, H // 2), jnp.int32),
            jax.ShapeDtypeStruct((1, H), jnp.float32),
            jax.ShapeDtypeStruct((1, H), jnp.float32),
        ],
    )(nf, we_s, we_r, g, we_g, be1, wn_g, bn1)


def _sc_gather_body(p_hbm, q_hbm, sidx_hbm, ridx_hbm, gp_out, gq_out,
                    sidx_v, ridx_v, bufp, bufq,
                    gspa, gsqa, gspb, gsqb, wspa, wsqa, wspb, wsqb):
    cid = lax.axis_index("c")
    sid = lax.axis_index("s")
    wid = sid * NC + cid
    base_row = wid * GNCH
    pltpu.sync_copy(sidx_hbm.at[wid], sidx_v)
    pltpu.sync_copy(ridx_hbm.at[wid], ridx_v)

    def fire_gathers(c0, s0, semp, semq):
        for b in range(GB):
            pltpu.async_copy(p_hbm.at[sidx_v.at[c0 + b]], bufp.at[s0 + b],
                             semp)
            pltpu.async_copy(q_hbm.at[ridx_v.at[c0 + b]], bufq.at[s0 + b],
                             semq)

    def drain_gathers(c0, s0, semp, semq):
        for b in range(GB):
            pltpu.make_async_copy(p_hbm.at[sidx_v.at[c0 + b]],
                                  bufp.at[s0 + b], semp).wait()
            pltpu.make_async_copy(q_hbm.at[ridx_v.at[c0 + b]],
                                  bufq.at[s0 + b], semq).wait()

    def fire_wbs(c0, s0, semp, semq):
        for b in range(GB):
            pltpu.async_copy(bufp.at[s0 + b], gp_out.at[base_row + c0 + b],
                             semp)
            pltpu.async_copy(bufq.at[s0 + b], gq_out.at[base_row + c0 + b],
                             semq)

    def drain_wbs(c0, s0, semp, semq):
        for b in range(GB):
            pltpu.make_async_copy(bufp.at[s0 + b],
                                  gp_out.at[base_row + c0 + b], semp).wait()
            pltpu.make_async_copy(bufq.at[s0 + b],
                                  gq_out.at[base_row + c0 + b], semq).wait()

    fire_gathers(0, 0, gspa, gsqa)

    def body(it, carry):
        base = it * 2 * GB

        @pl.when(it > 0)
        def _():
            drain_wbs(base - GB, GB, wspb, wsqb)

        fire_gathers(base + GB, GB, gspb, gsqb)
        drain_gathers(base, 0, gspa, gsqa)
        fire_wbs(base, 0, wspa, wsqa)

        @pl.when(it < GIT - 1)
        def _():
            drain_wbs(base, 0, wspa, wsqa)
            fire_gathers(base + 2 * GB, 0, gspa, gsqa)

        drain_gathers(base + GB, GB, gspb, gsqb)
        fire_wbs(base + GB, GB, wspb, wsqb)
        return carry

    lax.fori_loop(0, GIT, body, 0)
    last = (GIT - 1) * 2 * GB
    drain_wbs(last, 0, wspa, wsqa)
    drain_wbs(last + GB, GB, wspb, wsqb)


def _sc_gather(p, q, sidx3, ridx3):
    mesh = plsc.VectorSubcoreMesh(core_axis_name="c", subcore_axis_name="s")
    out = pl.kernel(
        _sc_gather_body,
        out_type=[
            jax.ShapeDtypeStruct((E // GCH, GCH, DN // 2), jnp.int32),
            jax.ShapeDtypeStruct((E // GCH, GCH, DN // 2), jnp.int32),
        ],
        mesh=mesh,
        scratch_types=[
            pltpu.VMEM((GNCH, GCH), jnp.int32),
            pltpu.VMEM((GNCH, GCH), jnp.int32),
            pltpu.VMEM((2 * GB, GCH, DN // 2), jnp.int32),
            pltpu.VMEM((2 * GB, GCH, DN // 2), jnp.int32),
        ] + [pltpu.SemaphoreType.DMA] * 8,
        compiler_params=pltpu.CompilerParams(use_tc_tiling_on_sc=False),
    )(p, q, sidx3, ridx3)
    return out


PK = 8
TE8 = TE // PK


def _edge_body(ef8, gp8, gq8, wlo, whi, vlo, vhi, clo8, chi8, be28,
               ne_out, e2g_out, acc):
    i = pl.program_id(0)
    plo, phi = _unpack_rows(gp8[...])
    qlo, qhi = _unpack_rows(gq8[...])
    efv = ef8[...]
    prelo = (jnp.dot(efv, wlo[...], preferred_element_type=jnp.float32)
             + plo + qlo + clo8[...])
    prehi = (jnp.dot(efv, whi[...], preferred_element_type=jnp.float32)
             + phi + qhi + chi8[...])
    glo = jnp.maximum(prelo, 0.0)
    ghi = jnp.maximum(prehi, 0.0)
    ne8 = (jnp.dot(glo, vlo[...], preferred_element_type=jnp.float32)
           + jnp.dot(ghi, vhi[...], preferred_element_type=jnp.float32)
           + be28[...])
    ne_out[...] = ne8
    part = jnp.sum(ne8, axis=0, keepdims=True)

    @pl.when(i == 0)
    def _():
        acc[...] = part

    @pl.when(i > 0)
    def _():
        acc[...] = acc[...] + part

    @pl.when(i == pl.num_programs(0) - 1)
    def _():
        a = acc[...]
        s = a[:, 0:DE]
        for k in range(1, PK):
            s = s + a[:, k * DE:(k + 1) * DE]
        e2g_out[...] = s


def _edge_mlp(ef8, gp8, gq8, wlo, whi, vlo, vhi, clo8, chi8, be28):
    grid = (E // TE,)
    full = lambda shape: pl.BlockSpec(shape, lambda i: (0, 0))
    return pl.pallas_call(
        _edge_body,
        grid=grid,
        in_specs=[
            pl.BlockSpec((TE8, PK * DE), lambda i: (i, 0)),
            pl.BlockSpec((TE8, PK * DN // 2), lambda i: (i, 0)),
            pl.BlockSpec((TE8, PK * DN // 2), lambda i: (i, 0)),
            full((PK * DE, PK * H // 2)), full((PK * DE, PK * H // 2)),
            full((PK * H // 2, PK * DE)), full((PK * H // 2, PK * DE)),
            full((1, PK * H // 2)), full((1, PK * H // 2)),
            full((1, PK * DE)),
        ],
        out_specs=[
            pl.BlockSpec((TE8, PK * DE), lambda i: (i, 0)),
            full((1, DE)),
        ],
        out_shape=[
            jax.ShapeDtypeStruct((E // PK, PK * DE), jnp.float32),
            jax.ShapeDtypeStruct((1, DE), jnp.float32),
        ],
        scratch_shapes=[pltpu.VMEM((1, PK * DE), jnp.float32)],
    )(ef8, gp8, gq8, wlo, whi, vlo, vhi, clo8, chi8, be28)


SBW = NBIG // 2 // NS


def _sc_scatter_body(ne_hbm, sidx_hbm, ridx_hbm, agg_out,
                     tabs, tabr, ne_buf, sidx_v, ridx_v,
                     sem_ne, sem_si, sem_ri, sem_add):
    cid = lax.axis_index("c")
    sid = lax.axis_index("s")

    zrow = jnp.zeros((DE,), jnp.float32)

    def zloop(i, c):
        ne_buf[0, i, :] = zrow
        return c

    lax.fori_loop(0, RPT, zloop, 0)
    pltpu.sync_copy(ne_buf.at[0, pl.ds(0, RPT)], tabs.at[pl.ds(sid * RPT, RPT)])
    pltpu.sync_copy(ne_buf.at[0, pl.ds(0, RPT)], tabr.at[pl.ds(sid * RPT, RPT)])
    plsc.subcore_barrier()

    base_big = cid * (NBIG // 2) + sid * SBW

    def fire_loads(t, p):
        pltpu.async_copy(ne_hbm.at[base_big + t], ne_buf.at[p], sem_ne)
        pltpu.async_copy(sidx_hbm.at[base_big + t], sidx_v.at[p], sem_si)
        pltpu.async_copy(ridx_hbm.at[base_big + t], ridx_v.at[p], sem_ri)

    def drain_loads(t, p):
        pltpu.make_async_copy(ne_hbm.at[base_big + t], ne_buf.at[p],
                              sem_ne).wait()
        pltpu.make_async_copy(sidx_hbm.at[base_big + t], sidx_v.at[p],
                              sem_si).wait()
        pltpu.make_async_copy(ridx_hbm.at[base_big + t], ridx_v.at[p],
                              sem_ri).wait()

    fire_loads(0, 0)

    def big(t, carry):
        p = lax.rem(t, 2)
        drain_loads(t, p)
        for j in range(NIN):
            pltpu.async_copy(ne_buf.at[p, pl.ds(j * CH, CH)],
                             tabs.at[sidx_v.at[p, j]], sem_add, add=True)
            pltpu.async_copy(ne_buf.at[p, pl.ds(j * CH, CH)],
                             tabr.at[ridx_v.at[p, j]], sem_add, add=True)

        @pl.when(t < SBW - 1)
        def _():
            fire_loads(t + 1, 1 - p)

        for j in range(NIN):
            pltpu.make_async_copy(ne_buf.at[p, pl.ds(j * CH, CH)],
                                  tabs.at[sidx_v.at[p, j]], sem_add).wait()
            pltpu.make_async_copy(ne_buf.at[p, pl.ds(j * CH, CH)],
                                  tabr.at[ridx_v.at[p, j]], sem_add).wait()
        return carry

    lax.fori_loop(0, SBW, big, 0)
    plsc.subcore_barrier()
    pltpu.sync_copy(tabs.at[pl.ds(sid * RPT, RPT)], agg_out.at[cid, 0, sid])
    pltpu.sync_copy(tabr.at[pl.ds(sid * RPT, RPT)], agg_out.at[cid, 1, sid])


def _sc_scatter(ne3, sidxb, ridxb):
    mesh = plsc.VectorSubcoreMesh(core_axis_name="c", subcore_axis_name="s")
    return pl.kernel(
        _sc_scatter_body,
        out_type=jax.ShapeDtypeStruct((2, 2, NS, RPT, DE), jnp.float32),
        mesh=mesh,
        scratch_types=[
            pltpu.VMEM_SHARED((NP, DE), jnp.float32),
            pltpu.VMEM_SHARED((NP, DE), jnp.float32),
            pltpu.VMEM((2, BIG, DE), jnp.float32),
            pltpu.VMEM((2, NIN, CH), jnp.int32),
            pltpu.VMEM((2, NIN, CH), jnp.int32),
        ] + [pltpu.SemaphoreType.DMA] * 4,
        compiler_params=pltpu.CompilerParams(use_tc_tiling_on_sc=False),
    )(ne3, sidxb, ridxb)


def _node_body(nf, ag0, ag1, ag2, ag3, cn, wn_n, wn_s, wn_r, wn2, bn2,
               g, e2g, wg_g, wg_n, wg_e, bg1, wg2, bg2,
               nn_out, ng_out, nacc):
    i = pl.program_id(0)
    ags = ag0[0] + ag1[0]
    agr = ag2[0] + ag3[0]
    pre = (jnp.dot(nf[...], wn_n[...], preferred_element_type=jnp.float32)
           + jnp.dot(ags, wn_s[...], preferred_element_type=jnp.float32)
           + jnp.dot(agr, wn_r[...], preferred_element_type=jnp.float32)
           + cn[...])
    hn = jnp.maximum(pre, 0.0)
    nn = jnp.dot(hn, wn2[...], preferred_element_type=jnp.float32) + bn2[...]
    nn_out[...] = nn
    part = jnp.sum(nn, axis=0, keepdims=True)

    @pl.when(i == 0)
    def _():
        nacc[...] = part

    @pl.when(i > 0)
    def _():
        nacc[...] = nacc[...] + part

    @pl.when(i == pl.num_programs(0) - 1)
    def _():
        gpre = (jnp.dot(g[...], wg_g[...], preferred_element_type=jnp.float32)
                + jnp.dot(nacc[...], wg_n[...],
                          preferred_element_type=jnp.float32)
                + jnp.dot(e2g[...], wg_e[...],
                          preferred_element_type=jnp.float32)
                + bg1[...])
        hg = jnp.maximum(gpre, 0.0)
        ng_out[...] = jnp.dot(hg, wg2[...],
                              preferred_element_type=jnp.float32) + bg2[...]


def _node_mlp(nf, agg2, cn, wn_n, wn_s, wn_r, wn2, bn2,
              g, e2g, wg_g, wg_n, wg_e, bg1, wg2, bg2):
    grid = (N // TN,)
    full = lambda shape: pl.BlockSpec(shape, lambda i: (0, 0))
    aspec = [pl.BlockSpec((1, TN, DE), lambda i, k=k: (k, i, 0))
             for k in (0, 2, 1, 3)]
    return pl.pallas_call(
        _node_body,
        grid=grid,
        in_specs=[
            pl.BlockSpec((TN, DN), lambda i: (i, 0)),
            *aspec,
            full((1, H)), full((DN, H)), full((DE, H)), full((DE, H)),
            full((H, DN)), full((1, DN)),
            full((1, DG)), full((1, DE)),
            full((DG, H)), full((DN, H)), full((DE, H)), full((1, H)),
            full((H, DG)), full((1, DG)),
        ],
        out_specs=[
            pl.BlockSpec((TN, DN), lambda i: (i, 0)),
            full((1, DG)),
        ],
        out_shape=[
            jax.ShapeDtypeStruct((N, DN), jnp.float32),
            jax.ShapeDtypeStruct((1, DG), jnp.float32),
        ],
        scratch_shapes=[pltpu.VMEM((1, DN), jnp.float32)],
    )(nf, agg2, agg2, agg2, agg2, cn, wn_n, wn_s, wn_r, wn2, bn2,
      g, e2g, wg_g, wg_n, wg_e, bg1, wg2, bg2)


def kernel(node_features, edge_features, global_features, senders, receivers,
           We1, be1, We2, be2, Wn1, bn1, Wn2, bn2, Wg1, bg1, Wg2, bg2):
    we_e = We1[0:DE]
    we_s = We1[DE:DE + DN]
    we_r = We1[DE + DN:DE + 2 * DN]
    we_g = We1[DE + 2 * DN:]
    wn_n = Wn1[0:DN]
    wn_s = Wn1[DN:DN + DE]
    wn_r = Wn1[DN + DE:DN + 2 * DE]
    wn_g = Wn1[DN + 2 * DE:]
    wg_g = Wg1[0:DG]
    wg_n = Wg1[DG:DG + DN]
    wg_e = Wg1[DG + DN:]

    p, q, ce, cn = _projections(
        node_features, we_s, we_r, global_features, we_g,
        be1.reshape(1, H), wn_g, bn1.reshape(1, H))

    sidx3 = senders.reshape(NW, GNCH, GCH)
    ridx3 = receivers.reshape(NW, GNCH, GCH)
    gp3, gq3 = _sc_gather(p, q, sidx3, ridx3)

    eye8 = jnp.eye(PK, dtype=jnp.float32)
    wlo = jnp.kron(eye8, we_e[:, :H // 2])
    whi = jnp.kron(eye8, we_e[:, H // 2:])
    vlo = jnp.kron(eye8, We2[:H // 2])
    vhi = jnp.kron(eye8, We2[H // 2:])
    clo8 = jnp.tile(ce[:, :H // 2], (1, PK))
    chi8 = jnp.tile(ce[:, H // 2:], (1, PK))
    be28 = jnp.tile(be2.reshape(1, DE), (1, PK))

    ne8, e2g = _edge_mlp(
        edge_features.reshape(E // PK, PK * DE),
        gp3.reshape(E // PK, PK * DN // 2),
        gq3.reshape(E // PK, PK * DN // 2),
        wlo, whi, vlo, vhi, clo8, chi8, be28)
    new_edges = ne8.reshape(E, DE)

    ne3 = new_edges.reshape(NBIG, BIG, DE)
    agg5 = _sc_scatter(ne3, senders.reshape(NBIG, NIN, CH),
                       receivers.reshape(NBIG, NIN, CH))
    agg2 = agg5.reshape(4, NP, DE)

    new_nodes, new_global = _node_mlp(
        node_features, agg2, cn, wn_n, wn_s, wn_r, Wn2,
        bn2.reshape(1, DN), global_features, e2g,
        wg_g, wg_n, wg_e, bg1.reshape(1, H), Wg2, bg2.reshape(1, DG))

    return (new_nodes, new_edges, new_global)
